# baseline (speedup 1.0000x reference)
import functools

import jax
import jax.numpy as jnp
from jax import lax
from jax.experimental import pallas as pl
from jax.experimental.pallas import tpu as pltpu
from jax.experimental.pallas import tpu_sc as plsc

N = 10000
E = 320000
EU = 160000
DIN = 128
DH = 256

NC = 2
NS = 16
LANES = 16

ROWS_BLK = 1000
HALF = DH // NC
NPAD = 10240
NPT = NPAD // NS
K = 80
JV = HALF // LANES

_MESH = plsc.VectorSubcoreMesh(core_axis_name="c", subcore_axis_name="s",
                               num_cores=NC, num_subcores=NS)
_SC_PARAMS = pltpu.CompilerParams(needs_layout_passes=False)


def _pre_matmul_body(z_ref, w_ref, b_ref, out_ref):
    out_ref[...] = (
        jnp.dot(z_ref[...], w_ref[...], preferred_element_type=jnp.float32)
        + b_ref[...]
    )


def _pre_matmul(z, wcat, bcat):
    grid = (N // ROWS_BLK,)
    return pl.pallas_call(
        _pre_matmul_body,
        grid=grid,
        in_specs=[
            pl.BlockSpec((ROWS_BLK, DIN), lambda i: (i, 0)),
            pl.BlockSpec((DIN, 4 * DH), lambda i: (0, 0)),
            pl.BlockSpec((1, 4 * DH), lambda i: (0, 0)),
        ],
        out_specs=pl.BlockSpec((ROWS_BLK, 4 * DH), lambda i: (i, 0)),
        out_shape=jax.ShapeDtypeStruct((N, 4 * DH), jnp.float32),
    )(z, wcat, bcat)


def _normalize_rows(x):
    n = jnp.sqrt(jnp.sum(x * x, axis=1, keepdims=True))
    return x / jnp.maximum(n, 1e-12)


def _post_body(z_ref, sf_ref, ot_ref, sf2_ref, os_ref,
               f1a_ref, f1b_ref, f1bias_ref, f2w_ref, f2bias_ref, out_ref):
    r = _normalize_rows(jax.nn.relu(sf_ref[...] + ot_ref[...]))
    g = _normalize_rows(jax.nn.relu(sf2_ref[...] + os_ref[...]))
    h1 = jax.nn.relu(
        jnp.dot(r, f1a_ref[...], preferred_element_type=jnp.float32)
        + jnp.dot(g, f1b_ref[...], preferred_element_type=jnp.float32)
        + f1bias_ref[...]
    )
    h = jnp.dot(h1, f2w_ref[...], preferred_element_type=jnp.float32) + f2bias_ref[...]
    out_ref[...] = _normalize_rows(z_ref[...] + jax.nn.relu(h))


def _post(z, self_f, out_t, self_f2, out_s, F1A, F1B, F1bias, F2W, F2bias):
    grid = (N // ROWS_BLK,)
    row_spec = lambda c: pl.BlockSpec((ROWS_BLK, c), lambda i: (i, 0))
    full_spec = lambda r, c: pl.BlockSpec((r, c), lambda i: (0, 0))
    return pl.pallas_call(
        _post_body,
        grid=grid,
        in_specs=[
            row_spec(DIN), row_spec(DH), row_spec(DH), row_spec(DH), row_spec(DH),
            full_spec(DH, DIN), full_spec(DH, DIN), full_spec(1, DIN),
            full_spec(DIN, DIN), full_spec(1, DIN),
        ],
        out_specs=pl.BlockSpec((ROWS_BLK, DIN), lambda i: (i, 0)),
        out_shape=jax.ShapeDtypeStruct((N, DIN), jnp.float32),
    )(z, self_f, out_t, self_f2, out_s, F1A, F1B, F1bias, F2W, F2bias)


EPT_A = EU // NS
NCHUNK_A = EPT_A // K


@functools.partial(
    pl.kernel,
    out_type=jax.ShapeDtypeStruct((NC, NPAD, HALF), jnp.float32),
    mesh=_MESH,
    compiler_params=_SC_PARAMS,
    scratch_types=[
        pltpu.VMEM_SHARED((NPAD, HALF), jnp.float32),
        pltpu.VMEM((3, K), jnp.int32),
        pltpu.VMEM((3, K), jnp.int32),
        pltpu.VMEM((2, K, HALF), jnp.float32),
        pltpu.VMEM((2, K, HALF), jnp.float32),
        pltpu.SemaphoreType.DMA((3,)),
        pltpu.SemaphoreType.DMA((2,)),
        pltpu.SemaphoreType.DMA((2,)),
        pltpu.SemaphoreType.DMA((2,)),
    ],
)
def _sc_temporal(src_hbm, dst_hbm, zwt_hbm, tf_hbm, zeros_hbm, out_hbm,
                 acc, sbuf, dbuf, rows, tfb, isem, gsem, tsem, ssem):
    c = lax.axis_index("c")
    s = lax.axis_index("s")

    pltpu.sync_copy(zeros_hbm, acc.at[pl.ds(s * NPT, NPT)])
    plsc.subcore_barrier()

    def idx_descs(k, b3):
        base = s * EPT_A + k * K
        return (
            pltpu.make_async_copy(src_hbm.at[pl.ds(base, K)], sbuf.at[b3],
                                  isem.at[b3]),
            pltpu.make_async_copy(dst_hbm.at[pl.ds(base, K)], dbuf.at[b3],
                                  isem.at[b3]),
        )

    def issue_idx(k):
        for d in idx_descs(k, lax.rem(k, 3)):
            d.start()

    def wait_idx(k):
        for d in idx_descs(k, lax.rem(k, 3)):
            d.wait()

    def gather_desc(k):
        b2 = lax.rem(k, 2)
        b3 = lax.rem(k, 3)
        return pltpu.make_async_copy(
            zwt_hbm.at[c].at[sbuf.at[b3]], rows.at[b2], gsem.at[b2])

    def tf_desc(k):
        b2 = lax.rem(k, 2)
        base = s * EPT_A + k * K
        return pltpu.make_async_copy(
            tf_hbm.at[pl.ds(base, K), pl.ds(c * HALF, HALF)],
            tfb.at[b2], tsem.at[b2])

    def scat_desc(k):
        b2 = lax.rem(k, 2)
        b3 = lax.rem(k, 3)
        return pltpu.make_async_copy(
            rows.at[b2], acc.at[dbuf.at[b3]], ssem.at[b2])

    issue_idx(0)
    wait_idx(0)
    gather_desc(0).start()
    tf_desc(0).start()
    issue_idx(1)

    def chunk(k, carry):
        b2 = lax.rem(k, 2)

        @pl.when(k >= 1)
        def _drain_scatter():
            scat_desc(k - 1).wait()

        @pl.when(k + 1 < NCHUNK_A)
        def _next_streams():
            wait_idx(k + 1)
            gather_desc(k + 1).start()
            tf_desc(k + 1).start()

        @pl.when(k + 2 < NCHUNK_A)
        def _next_idx():
            issue_idx(k + 2)

        gather_desc(k).wait()
        tf_desc(k).wait()

        def addloop(e, c2):
            for j in range(JV):
                sl = pl.ds(j * LANES, LANES)
                rows[b2, e, sl] = rows[b2, e, sl] + tfb[b2, e, sl]
            return c2

        lax.fori_loop(0, K, addloop, 0)
        scat_desc(k).start(add=True)
        return carry

    lax.fori_loop(0, NCHUNK_A, chunk, 0)
    scat_desc(NCHUNK_A - 1).wait()
    plsc.subcore_barrier()
    pltpu.sync_copy(acc.at[pl.ds(s * NPT, NPT)],
                    out_hbm.at[c].at[pl.ds(s * NPT, NPT)])


EPT_B = E // NS
NCHUNK_B = EPT_B // K


@functools.partial(
    pl.kernel,
    out_type=jax.ShapeDtypeStruct((NC, NPAD, HALF), jnp.float32),
    mesh=_MESH,
    compiler_params=_SC_PARAMS,
    scratch_types=[
        pltpu.VMEM_SHARED((NPAD, HALF), jnp.float32),
        pltpu.VMEM_SHARED((NPAD,), jnp.float32),
        pltpu.VMEM((NPAD,), jnp.float32),
        pltpu.VMEM((3, K), jnp.int32),
        pltpu.VMEM((3, K), jnp.int32),
        pltpu.VMEM((3, K), jnp.float32),
        pltpu.VMEM((2, K, HALF), jnp.float32),
        pltpu.SemaphoreType.DMA((3,)),
        pltpu.SemaphoreType.DMA((2,)),
        pltpu.SemaphoreType.DMA,
        pltpu.SemaphoreType.DMA((2,)),
    ],
)
def _sc_structural(row_hbm, col_hbm, td_hbm, zwn_hbm, zeros_hbm, zcol_hbm,
                   out_hbm, acc, tds, tdl, rbuf, cbuf, tbuf, rows,
                   isem, gsem, psem, ssem):
    c = lax.axis_index("c")
    s = lax.axis_index("s")

    pltpu.sync_copy(zeros_hbm, acc.at[pl.ds(s * NPT, NPT)])
    pltpu.sync_copy(zcol_hbm, tds.at[pl.ds(s * NPT, NPT)])
    plsc.subcore_barrier()

    def idx1_descs(k, b3):
        base = s * EPT_B + k * K
        return (
            pltpu.make_async_copy(row_hbm.at[pl.ds(base, K)], rbuf.at[b3],
                                  isem.at[b3]),
            pltpu.make_async_copy(td_hbm.at[pl.ds(base, K)], tbuf.at[b3],
                                  isem.at[b3]),
        )

    def issue_idx1(k):
        for d in idx1_descs(k, lax.rem(k, 3)):
            d.start()

    def wait_idx1(k):
        for d in idx1_descs(k, lax.rem(k, 3)):
            d.wait()

    def p1scat_desc(k):
        b3 = lax.rem(k, 3)
        return pltpu.make_async_copy(tbuf.at[b3], tds.at[rbuf.at[b3]], psem)

    issue_idx1(0)
    issue_idx1(1)

    def p1(k, carry):
        wait_idx1(k)
        p1scat_desc(k).start(add=True)

        @pl.when(k >= 1)
        def _drain_prev():
            p1scat_desc(k - 1).wait()

        @pl.when(k + 2 < NCHUNK_B)
        def _next_idx():
            issue_idx1(k + 2)

        return carry

    lax.fori_loop(0, NCHUNK_B, p1, 0)
    p1scat_desc(NCHUNK_B - 1).wait()
    plsc.subcore_barrier()

    pltpu.sync_copy(tds, tdl)

    def fix(i, carry):
        sl = pl.ds(i * LANES, LANES)
        v = tdl[sl]
        tdl[sl] = jnp.where(v == 0.0, 1.0, v)
        return carry

    lax.fori_loop(0, NPAD // LANES, fix, 0)

    def idx3_descs(k, b3):
        base = s * EPT_B + k * K
        return (
            pltpu.make_async_copy(row_hbm.at[pl.ds(base, K)], rbuf.at[b3],
                                  isem.at[b3]),
            pltpu.make_async_copy(col_hbm.at[pl.ds(base, K)], cbuf.at[b3],
                                  isem.at[b3]),
            pltpu.make_async_copy(td_hbm.at[pl.ds(base, K)], tbuf.at[b3],
                                  isem.at[b3]),
        )

    def issue_idx3(k):
        for d in idx3_descs(k, lax.rem(k, 3)):
            d.start()

    def wait_idx3(k):
        for d in idx3_descs(k, lax.rem(k, 3)):
            d.wait()

    def gather_desc(k):
        b2 = lax.rem(k, 2)
        b3 = lax.rem(k, 3)
        return pltpu.make_async_copy(
            zwn_hbm.at[c].at[rbuf.at[b3]], rows.at[b2], gsem.at[b2])

    def scat_desc(k):
        b2 = lax.rem(k, 2)
        b3 = lax.rem(k, 3)
        return pltpu.make_async_copy(
            rows.at[b2], acc.at[cbuf.at[b3]], ssem.at[b2])

    issue_idx3(0)
    wait_idx3(0)
    gather_desc(0).start()
    issue_idx3(1)

    def chunk3(k, carry):
        b3 = lax.rem(k, 3)

        @pl.when(k >= 1)
        def _drain_scatter():
            scat_desc(k - 1).wait()

        @pl.when(k + 1 < NCHUNK_B)
        def _next_gather():
            wait_idx3(k + 1)
            gather_desc(k + 1).start()

        @pl.when(k + 2 < NCHUNK_B)
        def _next_idx():
            issue_idx3(k + 2)

        def relw(j, c2):
            sl = pl.ds(j * LANES, LANES)
            den = plsc.load_gather(tdl, [rbuf[b3, sl]])
            tbuf[b3, sl] = tbuf[b3, sl] / den
            return c2

        lax.fori_loop(0, K // LANES, relw, 0)
        gather_desc(k).wait()
        b2 = lax.rem(k, 2)

        def scale(e, c2):
            w = plsc.load_gather(
                tbuf, [jnp.full((LANES,), b3, jnp.int32),
                       jnp.full((LANES,), e, jnp.int32)])
            for j in range(JV):
                sl = pl.ds(j * LANES, LANES)
                rows[b2, e, sl] = rows[b2, e, sl] * w
            return c2

        lax.fori_loop(0, K, scale, 0)
        scat_desc(k).start(add=True)
        return carry

    lax.fori_loop(0, NCHUNK_B, chunk3, 0)
    scat_desc(NCHUNK_B - 1).wait()
    plsc.subcore_barrier()
    pltpu.sync_copy(acc.at[pl.ds(s * NPT, NPT)],
                    out_hbm.at[c].at[pl.ds(s * NPT, NPT)])


def kernel(z, edge_index, temporal_features, time_diffs, unique_edges,
           WS1, bS1, WT, bT, WS2, bS2, WN, bN, F1W, F1b, F2W, F2b):
    wcat = jnp.concatenate([WS1, WT, WS2, WN], axis=1)
    bcat = jnp.concatenate([bS1, bT, bS2, bN]).reshape(1, 4 * DH)
    pre = _pre_matmul(z, wcat, bcat)
    self_f = pre[:, 0:DH]
    self_f2 = pre[:, 2 * DH:3 * DH]
    zwt2 = jnp.stack([pre[:, DH:DH + HALF], pre[:, DH + HALF:2 * DH]])
    zwn2 = jnp.stack([pre[:, 3 * DH:3 * DH + HALF], pre[:, 3 * DH + HALF:4 * DH]])

    zeros = jnp.zeros((NPT, HALF), jnp.float32)
    zcol = jnp.zeros((NPT,), jnp.float32)

    out_t2 = _sc_temporal(unique_edges[0], unique_edges[1], zwt2,
                          temporal_features, zeros)
    out_t = out_t2[:, :N, :].transpose(1, 0, 2).reshape(N, DH)

    td = time_diffs.astype(jnp.float32)
    out_s2 = _sc_structural(edge_index[0], edge_index[1], td, zwn2,
                            zeros, zcol)
    out_s = out_s2[:, :N, :].transpose(1, 0, 2).reshape(N, DH)

    F1A = F1W[:DH]
    F1B = F1W[DH:]
    return _post(z, self_f, out_t, self_f2, out_s,
                 F1A, F1B, F1b.reshape(1, DIN), F2W, F2b.reshape(1, DIN))


# speedup vs baseline: 1.4830x; 1.4830x over previous
import functools

import jax
import jax.numpy as jnp
from jax import lax
from jax.experimental import pallas as pl
from jax.experimental.pallas import tpu as pltpu
from jax.experimental.pallas import tpu_sc as plsc

N = 10000
E = 320000
EU = 160000
DIN = 128
DH = 256

NC = 2
NS = 16
LANES = 16

ROWS_BLK = 1000
HALF = DH // NC
NPAD = 10240
NPT = NPAD // NS
K = 80
JV = HALF // LANES

_MESH = plsc.VectorSubcoreMesh(core_axis_name="c", subcore_axis_name="s",
                               num_cores=NC, num_subcores=NS)
_SC_PARAMS = pltpu.CompilerParams(needs_layout_passes=False)


def _pre_matmul_body(z_ref, w_ref, b_ref, out_ref):
    out_ref[...] = (
        jnp.dot(z_ref[...], w_ref[...], preferred_element_type=jnp.float32)
        + b_ref[...]
    )


def _pre_matmul(z, wcat, bcat):
    grid = (N // ROWS_BLK,)
    return pl.pallas_call(
        _pre_matmul_body,
        grid=grid,
        in_specs=[
            pl.BlockSpec((ROWS_BLK, DIN), lambda i: (i, 0)),
            pl.BlockSpec((DIN, 4 * DH), lambda i: (0, 0)),
            pl.BlockSpec((1, 4 * DH), lambda i: (0, 0)),
        ],
        out_specs=pl.BlockSpec((ROWS_BLK, 4 * DH), lambda i: (i, 0)),
        out_shape=jax.ShapeDtypeStruct((N, 4 * DH), jnp.float32),
    )(z, wcat, bcat)


def _normalize_rows(x):
    n = jnp.sqrt(jnp.sum(x * x, axis=1, keepdims=True))
    return x / jnp.maximum(n, 1e-12)


def _post_body(z_ref, sf_ref, ot_ref, sf2_ref, os_ref,
               f1a_ref, f1b_ref, f1bias_ref, f2w_ref, f2bias_ref, out_ref):
    r = _normalize_rows(jax.nn.relu(sf_ref[...] + ot_ref[...]))
    g = _normalize_rows(jax.nn.relu(sf2_ref[...] + os_ref[...]))
    h1 = jax.nn.relu(
        jnp.dot(r, f1a_ref[...], preferred_element_type=jnp.float32)
        + jnp.dot(g, f1b_ref[...], preferred_element_type=jnp.float32)
        + f1bias_ref[...]
    )
    h = jnp.dot(h1, f2w_ref[...], preferred_element_type=jnp.float32) + f2bias_ref[...]
    out_ref[...] = _normalize_rows(z_ref[...] + jax.nn.relu(h))


def _post(z, self_f, out_t, self_f2, out_s, F1A, F1B, F1bias, F2W, F2bias):
    grid = (N // ROWS_BLK,)
    row_spec = lambda c: pl.BlockSpec((ROWS_BLK, c), lambda i: (i, 0))
    full_spec = lambda r, c: pl.BlockSpec((r, c), lambda i: (0, 0))
    return pl.pallas_call(
        _post_body,
        grid=grid,
        in_specs=[
            row_spec(DIN), row_spec(DH), row_spec(DH), row_spec(DH), row_spec(DH),
            full_spec(DH, DIN), full_spec(DH, DIN), full_spec(1, DIN),
            full_spec(DIN, DIN), full_spec(1, DIN),
        ],
        out_specs=pl.BlockSpec((ROWS_BLK, DIN), lambda i: (i, 0)),
        out_shape=jax.ShapeDtypeStruct((N, DIN), jnp.float32),
    )(z, self_f, out_t, self_f2, out_s, F1A, F1B, F1bias, F2W, F2bias)


EPT_A = EU // NS
NCHUNK_A = EPT_A // K


@functools.partial(
    pl.kernel,
    out_type=jax.ShapeDtypeStruct((NC, NPAD, HALF), jnp.float32),
    mesh=_MESH,
    compiler_params=_SC_PARAMS,
    scratch_types=[
        pltpu.VMEM_SHARED((NPAD, HALF), jnp.float32),
        pltpu.VMEM((3, K), jnp.int32),
        pltpu.VMEM((3, K), jnp.int32),
        pltpu.VMEM((2, K, HALF), jnp.float32),
        pltpu.VMEM((2, K, HALF), jnp.float32),
        pltpu.SemaphoreType.DMA((3,)),
        pltpu.SemaphoreType.DMA((2,)),
        pltpu.SemaphoreType.DMA((2,)),
        pltpu.SemaphoreType.DMA((2,)),
        pltpu.SemaphoreType.DMA((2,)),
    ],
)
def _sc_temporal(src_hbm, dst_hbm, zwt_hbm, tf_hbm, zeros_hbm, out_hbm,
                 acc, sbuf, dbuf, rows, tfb, isem, gsem, tsem, ssem, tssem):
    c = lax.axis_index("c")
    s = lax.axis_index("s")

    pltpu.sync_copy(zeros_hbm, acc.at[pl.ds(s * NPT, NPT)])
    plsc.subcore_barrier()

    def idx_descs(k, b3):
        base = s * EPT_A + k * K
        return (
            pltpu.make_async_copy(src_hbm.at[pl.ds(base, K)], sbuf.at[b3],
                                  isem.at[b3]),
            pltpu.make_async_copy(dst_hbm.at[pl.ds(base, K)], dbuf.at[b3],
                                  isem.at[b3]),
        )

    def issue_idx(k):
        for d in idx_descs(k, lax.rem(k, 3)):
            d.start()

    def wait_idx(k):
        for d in idx_descs(k, lax.rem(k, 3)):
            d.wait()

    def gather_desc(k):
        b2 = lax.rem(k, 2)
        b3 = lax.rem(k, 3)
        return pltpu.make_async_copy(
            zwt_hbm.at[c].at[sbuf.at[b3]], rows.at[b2], gsem.at[b2])

    def tf_desc(k):
        b2 = lax.rem(k, 2)
        base = s * EPT_A + k * K
        return pltpu.make_async_copy(
            tf_hbm.at[pl.ds(base, K), pl.ds(c * HALF, HALF)],
            tfb.at[b2], tsem.at[b2])

    def scat_desc(k):
        b2 = lax.rem(k, 2)
        b3 = lax.rem(k, 3)
        return pltpu.make_async_copy(
            rows.at[b2], acc.at[dbuf.at[b3]], ssem.at[b2])

    def scat_tf_desc(k):
        b2 = lax.rem(k, 2)
        b3 = lax.rem(k, 3)
        return pltpu.make_async_copy(
            tfb.at[b2], acc.at[dbuf.at[b3]], tssem.at[b2])

    issue_idx(0)
    wait_idx(0)
    gather_desc(0).start()
    tf_desc(0).start()
    issue_idx(1)

    def chunk(k, carry):
        b2 = lax.rem(k, 2)

        @pl.when(k >= 1)
        def _drain_scatter():
            scat_desc(k - 1).wait()
            scat_tf_desc(k - 1).wait()

        @pl.when(k + 1 < NCHUNK_A)
        def _next_streams():
            wait_idx(k + 1)
            gather_desc(k + 1).start()
            tf_desc(k + 1).start()

        @pl.when(k + 2 < NCHUNK_A)
        def _next_idx():
            issue_idx(k + 2)

        gather_desc(k).wait()
        scat_desc(k).start(add=True)
        tf_desc(k).wait()
        scat_tf_desc(k).start(add=True)
        return carry

    lax.fori_loop(0, NCHUNK_A, chunk, 0)
    scat_desc(NCHUNK_A - 1).wait()
    scat_tf_desc(NCHUNK_A - 1).wait()
    plsc.subcore_barrier()
    pltpu.sync_copy(acc.at[pl.ds(s * NPT, NPT)],
                    out_hbm.at[c].at[pl.ds(s * NPT, NPT)])


EPT_B = E // NS
NCHUNK_B = EPT_B // K


@functools.partial(
    pl.kernel,
    out_type=jax.ShapeDtypeStruct((NC, NPAD, HALF), jnp.float32),
    mesh=_MESH,
    compiler_params=_SC_PARAMS,
    scratch_types=[
        pltpu.VMEM_SHARED((NPAD, HALF), jnp.float32),
        pltpu.VMEM_SHARED((NPAD,), jnp.float32),
        pltpu.VMEM((NPAD,), jnp.float32),
        pltpu.VMEM((3, K), jnp.int32),
        pltpu.VMEM((3, K), jnp.int32),
        pltpu.VMEM((3, K), jnp.float32),
        pltpu.VMEM((2, K, HALF), jnp.float32),
        pltpu.SemaphoreType.DMA((3,)),
        pltpu.SemaphoreType.DMA((2,)),
        pltpu.SemaphoreType.DMA,
        pltpu.SemaphoreType.DMA((2,)),
    ],
)
def _sc_structural(row_hbm, col_hbm, td_hbm, zwn_hbm, zeros_hbm, zcol_hbm,
                   out_hbm, acc, tds, tdl, rbuf, cbuf, tbuf, rows,
                   isem, gsem, psem, ssem):
    c = lax.axis_index("c")
    s = lax.axis_index("s")

    pltpu.sync_copy(zeros_hbm, acc.at[pl.ds(s * NPT, NPT)])
    pltpu.sync_copy(zcol_hbm, tds.at[pl.ds(s * NPT, NPT)])
    plsc.subcore_barrier()

    def idx1_descs(k, b3):
        base = s * EPT_B + k * K
        return (
            pltpu.make_async_copy(row_hbm.at[pl.ds(base, K)], rbuf.at[b3],
                                  isem.at[b3]),
            pltpu.make_async_copy(td_hbm.at[pl.ds(base, K)], tbuf.at[b3],
                                  isem.at[b3]),
        )

    def issue_idx1(k):
        for d in idx1_descs(k, lax.rem(k, 3)):
            d.start()

    def wait_idx1(k):
        for d in idx1_descs(k, lax.rem(k, 3)):
            d.wait()

    def p1scat_desc(k):
        b3 = lax.rem(k, 3)
        return pltpu.make_async_copy(tbuf.at[b3], tds.at[rbuf.at[b3]], psem)

    issue_idx1(0)
    issue_idx1(1)

    def p1(k, carry):
        wait_idx1(k)
        p1scat_desc(k).start(add=True)

        @pl.when(k >= 1)
        def _drain_prev():
            p1scat_desc(k - 1).wait()

        @pl.when(k + 2 < NCHUNK_B)
        def _next_idx():
            issue_idx1(k + 2)

        return carry

    lax.fori_loop(0, NCHUNK_B, p1, 0)
    p1scat_desc(NCHUNK_B - 1).wait()
    plsc.subcore_barrier()

    pltpu.sync_copy(tds, tdl)

    def fix(i, carry):
        sl = pl.ds(i * LANES, LANES)
        v = tdl[sl]
        tdl[sl] = jnp.where(v == 0.0, 1.0, v)
        return carry

    lax.fori_loop(0, NPAD // LANES, fix, 0)

    def idx3_descs(k, b3):
        base = s * EPT_B + k * K
        return (
            pltpu.make_async_copy(row_hbm.at[pl.ds(base, K)], rbuf.at[b3],
                                  isem.at[b3]),
            pltpu.make_async_copy(col_hbm.at[pl.ds(base, K)], cbuf.at[b3],
                                  isem.at[b3]),
            pltpu.make_async_copy(td_hbm.at[pl.ds(base, K)], tbuf.at[b3],
                                  isem.at[b3]),
        )

    def issue_idx3(k):
        for d in idx3_descs(k, lax.rem(k, 3)):
            d.start()

    def wait_idx3(k):
        for d in idx3_descs(k, lax.rem(k, 3)):
            d.wait()

    def gather_desc(k):
        b2 = lax.rem(k, 2)
        b3 = lax.rem(k, 3)
        return pltpu.make_async_copy(
            zwn_hbm.at[c].at[rbuf.at[b3]], rows.at[b2], gsem.at[b2])

    def scat_desc(k):
        b2 = lax.rem(k, 2)
        b3 = lax.rem(k, 3)
        return pltpu.make_async_copy(
            rows.at[b2], acc.at[cbuf.at[b3]], ssem.at[b2])

    issue_idx3(0)
    wait_idx3(0)
    gather_desc(0).start()
    issue_idx3(1)

    def chunk3(k, carry):
        b3 = lax.rem(k, 3)

        @pl.when(k >= 1)
        def _drain_scatter():
            scat_desc(k - 1).wait()

        @pl.when(k + 1 < NCHUNK_B)
        def _next_gather():
            wait_idx3(k + 1)
            gather_desc(k + 1).start()

        @pl.when(k + 2 < NCHUNK_B)
        def _next_idx():
            issue_idx3(k + 2)

        def relw(j, c2):
            sl = pl.ds(j * LANES, LANES)
            den = plsc.load_gather(tdl, [rbuf[b3, sl]])
            tbuf[b3, sl] = tbuf[b3, sl] / den
            return c2

        lax.fori_loop(0, K // LANES, relw, 0)
        gather_desc(k).wait()
        b2 = lax.rem(k, 2)

        def scale(e, c2):
            w = plsc.load_gather(
                tbuf, [jnp.full((LANES,), b3, jnp.int32),
                       jnp.full((LANES,), e, jnp.int32)])
            for j in range(JV):
                sl = pl.ds(j * LANES, LANES)
                rows[b2, e, sl] = rows[b2, e, sl] * w
            return c2

        lax.fori_loop(0, K, scale, 0)
        scat_desc(k).start(add=True)
        return carry

    lax.fori_loop(0, NCHUNK_B, chunk3, 0)
    scat_desc(NCHUNK_B - 1).wait()
    plsc.subcore_barrier()
    pltpu.sync_copy(acc.at[pl.ds(s * NPT, NPT)],
                    out_hbm.at[c].at[pl.ds(s * NPT, NPT)])


def kernel(z, edge_index, temporal_features, time_diffs, unique_edges,
           WS1, bS1, WT, bT, WS2, bS2, WN, bN, F1W, F1b, F2W, F2b):
    wcat = jnp.concatenate([WS1, WT, WS2, WN], axis=1)
    bcat = jnp.concatenate([bS1, bT, bS2, bN]).reshape(1, 4 * DH)
    pre = _pre_matmul(z, wcat, bcat)
    self_f = pre[:, 0:DH]
    self_f2 = pre[:, 2 * DH:3 * DH]
    zwt2 = jnp.stack([pre[:, DH:DH + HALF], pre[:, DH + HALF:2 * DH]])
    zwn2 = jnp.stack([pre[:, 3 * DH:3 * DH + HALF], pre[:, 3 * DH + HALF:4 * DH]])

    zeros = jnp.zeros((NPT, HALF), jnp.float32)
    zcol = jnp.zeros((NPT,), jnp.float32)

    out_t2 = _sc_temporal(unique_edges[0], unique_edges[1], zwt2,
                          temporal_features, zeros)
    out_t = out_t2[:, :N, :].transpose(1, 0, 2).reshape(N, DH)

    td = time_diffs.astype(jnp.float32)
    out_s2 = _sc_structural(edge_index[0], edge_index[1], td, zwn2,
                            zeros, zcol)
    out_s = out_s2[:, :N, :].transpose(1, 0, 2).reshape(N, DH)

    F1A = F1W[:DH]
    F1B = F1W[DH:]
    return _post(z, self_f, out_t, self_f2, out_s,
                 F1A, F1B, F1b.reshape(1, DIN), F2W, F2b.reshape(1, DIN))


# speedup vs baseline: 1.6769x; 1.1307x over previous
import functools

import jax
import jax.numpy as jnp
from jax import lax
from jax.experimental import pallas as pl
from jax.experimental.pallas import tpu as pltpu
from jax.experimental.pallas import tpu_sc as plsc

N = 10000
E = 320000
EU = 160000
DIN = 128
DH = 256

NC = 2
NS = 16
LANES = 16

ROWS_BLK = 1000
HALF = DH // NC
NPAD = 10240
NPT = NPAD // NS
K = 80
JV = HALF // LANES

_MESH = plsc.VectorSubcoreMesh(core_axis_name="c", subcore_axis_name="s",
                               num_cores=NC, num_subcores=NS)
_SC_PARAMS = pltpu.CompilerParams(needs_layout_passes=False)


def _pre_matmul_body(z_ref, w_ref, b_ref, out_ref):
    out_ref[...] = (
        jnp.dot(z_ref[...], w_ref[...], preferred_element_type=jnp.float32)
        + b_ref[...]
    )


def _pre_matmul(z, wcat, bcat):
    grid = (N // ROWS_BLK,)
    return pl.pallas_call(
        _pre_matmul_body,
        grid=grid,
        in_specs=[
            pl.BlockSpec((ROWS_BLK, DIN), lambda i: (i, 0)),
            pl.BlockSpec((DIN, 4 * DH), lambda i: (0, 0)),
            pl.BlockSpec((1, 4 * DH), lambda i: (0, 0)),
        ],
        out_specs=pl.BlockSpec((ROWS_BLK, 4 * DH), lambda i: (i, 0)),
        out_shape=jax.ShapeDtypeStruct((N, 4 * DH), jnp.float32),
    )(z, wcat, bcat)


def _normalize_rows(x):
    n = jnp.sqrt(jnp.sum(x * x, axis=1, keepdims=True))
    return x / jnp.maximum(n, 1e-12)


def _post_body(z_ref, sf_ref, ot_ref, sf2_ref, os_ref,
               f1a_ref, f1b_ref, f1bias_ref, f2w_ref, f2bias_ref, out_ref):
    r = _normalize_rows(jax.nn.relu(sf_ref[...] + ot_ref[...]))
    g = _normalize_rows(jax.nn.relu(sf2_ref[...] + os_ref[...]))
    h1 = jax.nn.relu(
        jnp.dot(r, f1a_ref[...], preferred_element_type=jnp.float32)
        + jnp.dot(g, f1b_ref[...], preferred_element_type=jnp.float32)
        + f1bias_ref[...]
    )
    h = jnp.dot(h1, f2w_ref[...], preferred_element_type=jnp.float32) + f2bias_ref[...]
    out_ref[...] = _normalize_rows(z_ref[...] + jax.nn.relu(h))


def _post(z, self_f, out_t, self_f2, out_s, F1A, F1B, F1bias, F2W, F2bias):
    grid = (N // ROWS_BLK,)
    row_spec = lambda c: pl.BlockSpec((ROWS_BLK, c), lambda i: (i, 0))
    full_spec = lambda r, c: pl.BlockSpec((r, c), lambda i: (0, 0))
    return pl.pallas_call(
        _post_body,
        grid=grid,
        in_specs=[
            row_spec(DIN), row_spec(DH), row_spec(DH), row_spec(DH), row_spec(DH),
            full_spec(DH, DIN), full_spec(DH, DIN), full_spec(1, DIN),
            full_spec(DIN, DIN), full_spec(1, DIN),
        ],
        out_specs=pl.BlockSpec((ROWS_BLK, DIN), lambda i: (i, 0)),
        out_shape=jax.ShapeDtypeStruct((N, DIN), jnp.float32),
    )(z, self_f, out_t, self_f2, out_s, F1A, F1B, F1bias, F2W, F2bias)


EPT_A = EU // NS
NCHUNK_A = EPT_A // K


@functools.partial(
    pl.kernel,
    out_type=jax.ShapeDtypeStruct((NC, NPAD, HALF), jnp.float32),
    mesh=_MESH,
    compiler_params=_SC_PARAMS,
    scratch_types=[
        pltpu.VMEM_SHARED((NPAD, HALF), jnp.float32),
        pltpu.VMEM((3, K), jnp.int32),
        pltpu.VMEM((3, K), jnp.int32),
        pltpu.VMEM((2, K, HALF), jnp.float32),
        pltpu.VMEM((2, K, HALF), jnp.float32),
        pltpu.SemaphoreType.DMA((3,)),
        pltpu.SemaphoreType.DMA((2,)),
        pltpu.SemaphoreType.DMA((2,)),
        pltpu.SemaphoreType.DMA((2,)),
        pltpu.SemaphoreType.DMA((2,)),
    ],
)
def _sc_temporal(src_hbm, dst_hbm, zwt_hbm, tf_hbm, zeros_hbm, out_hbm,
                 acc, sbuf, dbuf, rows, tfb, isem, gsem, tsem, ssem, tssem):
    c = lax.axis_index("c")
    s = lax.axis_index("s")

    pltpu.sync_copy(zeros_hbm, acc.at[pl.ds(s * NPT, NPT)])
    plsc.subcore_barrier()

    def idx_descs(k, b3):
        base = s * EPT_A + k * K
        return (
            pltpu.make_async_copy(src_hbm.at[pl.ds(base, K)], sbuf.at[b3],
                                  isem.at[b3]),
            pltpu.make_async_copy(dst_hbm.at[pl.ds(base, K)], dbuf.at[b3],
                                  isem.at[b3]),
        )

    def issue_idx(k):
        for d in idx_descs(k, lax.rem(k, 3)):
            d.start()

    def wait_idx(k):
        for d in idx_descs(k, lax.rem(k, 3)):
            d.wait()

    def gather_desc(k):
        b2 = lax.rem(k, 2)
        b3 = lax.rem(k, 3)
        return pltpu.make_async_copy(
            zwt_hbm.at[c].at[sbuf.at[b3]], rows.at[b2], gsem.at[b2])

    def tf_desc(k):
        b2 = lax.rem(k, 2)
        base = s * EPT_A + k * K
        return pltpu.make_async_copy(
            tf_hbm.at[pl.ds(base, K), pl.ds(c * HALF, HALF)],
            tfb.at[b2], tsem.at[b2])

    def scat_desc(k):
        b2 = lax.rem(k, 2)
        b3 = lax.rem(k, 3)
        return pltpu.make_async_copy(
            rows.at[b2], acc.at[dbuf.at[b3]], ssem.at[b2])

    def scat_tf_desc(k):
        b2 = lax.rem(k, 2)
        b3 = lax.rem(k, 3)
        return pltpu.make_async_copy(
            tfb.at[b2], acc.at[dbuf.at[b3]], tssem.at[b2])

    issue_idx(0)
    wait_idx(0)
    gather_desc(0).start()
    tf_desc(0).start()
    issue_idx(1)

    def chunk(k, carry):
        b2 = lax.rem(k, 2)

        @pl.when(k >= 1)
        def _drain_scatter():
            scat_desc(k - 1).wait()
            scat_tf_desc(k - 1).wait()

        @pl.when(k + 1 < NCHUNK_A)
        def _next_streams():
            wait_idx(k + 1)
            gather_desc(k + 1).start()
            tf_desc(k + 1).start()

        @pl.when(k + 2 < NCHUNK_A)
        def _next_idx():
            issue_idx(k + 2)

        gather_desc(k).wait()
        scat_desc(k).start(add=True)
        tf_desc(k).wait()
        scat_tf_desc(k).start(add=True)
        return carry

    lax.fori_loop(0, NCHUNK_A, chunk, 0)
    scat_desc(NCHUNK_A - 1).wait()
    scat_tf_desc(NCHUNK_A - 1).wait()
    plsc.subcore_barrier()
    pltpu.sync_copy(acc.at[pl.ds(s * NPT, NPT)],
                    out_hbm.at[c].at[pl.ds(s * NPT, NPT)])


EPT_B = E // NS
NCHUNK_B = EPT_B // K


@functools.partial(
    pl.kernel,
    out_type=jax.ShapeDtypeStruct((NC, NPAD, HALF), jnp.float32),
    mesh=_MESH,
    compiler_params=_SC_PARAMS,
    scratch_types=[
        pltpu.VMEM_SHARED((NPAD, HALF), jnp.float32),
        pltpu.VMEM_SHARED((NPAD,), jnp.float32),
        pltpu.VMEM((NPAD,), jnp.float32),
        pltpu.VMEM((3, K), jnp.int32),
        pltpu.VMEM((3, K), jnp.int32),
        pltpu.VMEM((3, K), jnp.float32),
        pltpu.VMEM((2, K, HALF), jnp.float32),
        pltpu.SemaphoreType.DMA((3,)),
        pltpu.SemaphoreType.DMA((2,)),
        pltpu.SemaphoreType.DMA,
        pltpu.SemaphoreType.DMA((2,)),
    ],
)
def _sc_structural(row_hbm, col_hbm, td_hbm, zwn_hbm, zeros_hbm, zcol_hbm,
                   out_hbm, acc, tds, tdl, rbuf, cbuf, tbuf, rows,
                   isem, gsem, psem, ssem):
    c = lax.axis_index("c")
    s = lax.axis_index("s")

    pltpu.sync_copy(zeros_hbm, acc.at[pl.ds(s * NPT, NPT)])
    pltpu.sync_copy(zcol_hbm, tds.at[pl.ds(s * NPT, NPT)])
    plsc.subcore_barrier()

    def idx1_descs(k, b3):
        base = s * EPT_B + k * K
        return (
            pltpu.make_async_copy(row_hbm.at[pl.ds(base, K)], rbuf.at[b3],
                                  isem.at[b3]),
            pltpu.make_async_copy(td_hbm.at[pl.ds(base, K)], tbuf.at[b3],
                                  isem.at[b3]),
        )

    def issue_idx1(k):
        for d in idx1_descs(k, lax.rem(k, 3)):
            d.start()

    def wait_idx1(k):
        for d in idx1_descs(k, lax.rem(k, 3)):
            d.wait()

    def p1scat_desc(k):
        b3 = lax.rem(k, 3)
        return pltpu.make_async_copy(tbuf.at[b3], tds.at[rbuf.at[b3]], psem)

    issue_idx1(0)
    issue_idx1(1)

    def p1(k, carry):
        wait_idx1(k)
        p1scat_desc(k).start(add=True)

        @pl.when(k >= 1)
        def _drain_prev():
            p1scat_desc(k - 1).wait()

        @pl.when(k + 2 < NCHUNK_B)
        def _next_idx():
            issue_idx1(k + 2)

        return carry

    lax.fori_loop(0, NCHUNK_B, p1, 0)
    p1scat_desc(NCHUNK_B - 1).wait()
    plsc.subcore_barrier()

    pltpu.sync_copy(tds, tdl)

    def fix(i, carry):
        sl = pl.ds(i * LANES, LANES)
        v = tdl[sl]
        tdl[sl] = jnp.where(v == 0.0, 1.0, v)
        return carry

    lax.fori_loop(0, NPAD // LANES, fix, 0)

    def idx3_descs(k, b3):
        base = s * EPT_B + k * K
        return (
            pltpu.make_async_copy(row_hbm.at[pl.ds(base, K)], rbuf.at[b3],
                                  isem.at[b3]),
            pltpu.make_async_copy(col_hbm.at[pl.ds(base, K)], cbuf.at[b3],
                                  isem.at[b3]),
            pltpu.make_async_copy(td_hbm.at[pl.ds(base, K)], tbuf.at[b3],
                                  isem.at[b3]),
        )

    def issue_idx3(k):
        for d in idx3_descs(k, lax.rem(k, 3)):
            d.start()

    def wait_idx3(k):
        for d in idx3_descs(k, lax.rem(k, 3)):
            d.wait()

    def gather_desc(k):
        b2 = lax.rem(k, 2)
        b3 = lax.rem(k, 3)
        return pltpu.make_async_copy(
            zwn_hbm.at[c].at[rbuf.at[b3]], rows.at[b2], gsem.at[b2])

    def scat_desc(k):
        b2 = lax.rem(k, 2)
        b3 = lax.rem(k, 3)
        return pltpu.make_async_copy(
            rows.at[b2], acc.at[cbuf.at[b3]], ssem.at[b2])

    issue_idx3(0)
    wait_idx3(0)
    gather_desc(0).start()
    issue_idx3(1)

    def chunk3(k, carry):
        b3 = lax.rem(k, 3)

        @pl.when(k >= 1)
        def _drain_scatter():
            scat_desc(k - 1).wait()

        @pl.when(k + 1 < NCHUNK_B)
        def _next_gather():
            wait_idx3(k + 1)
            gather_desc(k + 1).start()

        @pl.when(k + 2 < NCHUNK_B)
        def _next_idx():
            issue_idx3(k + 2)

        def relw(j, c2):
            sl = pl.ds(j * LANES, LANES)
            den = plsc.load_gather(tdl, [rbuf[b3, sl]])
            tbuf[b3, sl] = tbuf[b3, sl] / den
            return c2

        lax.fori_loop(0, K // LANES, relw, 0)
        gather_desc(k).wait()
        b2 = lax.rem(k, 2)

        @plsc.parallel_loop(0, K, unroll=4)
        def scale(e):
            w = plsc.load_gather(
                tbuf, [jnp.full((LANES,), b3, jnp.int32),
                       jnp.full((LANES,), e, jnp.int32)])
            for j in range(JV):
                sl = pl.ds(j * LANES, LANES)
                rows[b2, e, sl] = rows[b2, e, sl] * w
        scat_desc(k).start(add=True)
        return carry

    lax.fori_loop(0, NCHUNK_B, chunk3, 0)
    scat_desc(NCHUNK_B - 1).wait()
    plsc.subcore_barrier()
    pltpu.sync_copy(acc.at[pl.ds(s * NPT, NPT)],
                    out_hbm.at[c].at[pl.ds(s * NPT, NPT)])


def kernel(z, edge_index, temporal_features, time_diffs, unique_edges,
           WS1, bS1, WT, bT, WS2, bS2, WN, bN, F1W, F1b, F2W, F2b):
    wcat = jnp.concatenate([WS1, WT, WS2, WN], axis=1)
    bcat = jnp.concatenate([bS1, bT, bS2, bN]).reshape(1, 4 * DH)
    pre = _pre_matmul(z, wcat, bcat)
    self_f = pre[:, 0:DH]
    self_f2 = pre[:, 2 * DH:3 * DH]
    zwt2 = jnp.stack([pre[:, DH:DH + HALF], pre[:, DH + HALF:2 * DH]])
    zwn2 = jnp.stack([pre[:, 3 * DH:3 * DH + HALF], pre[:, 3 * DH + HALF:4 * DH]])

    zeros = jnp.zeros((NPT, HALF), jnp.float32)
    zcol = jnp.zeros((NPT,), jnp.float32)

    out_t2 = _sc_temporal(unique_edges[0], unique_edges[1], zwt2,
                          temporal_features, zeros)
    out_t = out_t2[:, :N, :].transpose(1, 0, 2).reshape(N, DH)

    td = time_diffs.astype(jnp.float32)
    out_s2 = _sc_structural(edge_index[0], edge_index[1], td, zwn2,
                            zeros, zcol)
    out_s = out_s2[:, :N, :].transpose(1, 0, 2).reshape(N, DH)

    F1A = F1W[:DH]
    F1B = F1W[DH:]
    return _post(z, self_f, out_t, self_f2, out_s,
                 F1A, F1B, F1b.reshape(1, DIN), F2W, F2b.reshape(1, DIN))


# speedup vs baseline: 1.6775x; 1.0004x over previous
import functools

import jax
import jax.numpy as jnp
from jax import lax
from jax.experimental import pallas as pl
from jax.experimental.pallas import tpu as pltpu
from jax.experimental.pallas import tpu_sc as plsc

N = 10000
E = 320000
EU = 160000
DIN = 128
DH = 256

NC = 2
NS = 16
LANES = 16

ROWS_BLK = 1000
HALF = DH // NC
NPAD = 10240
NPT = NPAD // NS
K = 80
JV = HALF // LANES

_MESH = plsc.VectorSubcoreMesh(core_axis_name="c", subcore_axis_name="s",
                               num_cores=NC, num_subcores=NS)
_SC_PARAMS = pltpu.CompilerParams(needs_layout_passes=False)


def _pre_matmul_body(z_ref, w_ref, b_ref, out_ref):
    out_ref[...] = (
        jnp.dot(z_ref[...], w_ref[...], preferred_element_type=jnp.float32)
        + b_ref[...]
    )


def _pre_matmul(z, wcat, bcat):
    grid = (N // ROWS_BLK,)
    return pl.pallas_call(
        _pre_matmul_body,
        grid=grid,
        in_specs=[
            pl.BlockSpec((ROWS_BLK, DIN), lambda i: (i, 0)),
            pl.BlockSpec((DIN, 4 * DH), lambda i: (0, 0)),
            pl.BlockSpec((1, 4 * DH), lambda i: (0, 0)),
        ],
        out_specs=pl.BlockSpec((ROWS_BLK, 4 * DH), lambda i: (i, 0)),
        out_shape=jax.ShapeDtypeStruct((N, 4 * DH), jnp.float32),
    )(z, wcat, bcat)


def _normalize_rows(x):
    n = jnp.sqrt(jnp.sum(x * x, axis=1, keepdims=True))
    return x / jnp.maximum(n, 1e-12)


def _post_body(z_ref, sf_ref, ot_ref, sf2_ref, os_ref,
               f1a_ref, f1b_ref, f1bias_ref, f2w_ref, f2bias_ref, out_ref):
    r = _normalize_rows(jax.nn.relu(sf_ref[...] + ot_ref[...]))
    g = _normalize_rows(jax.nn.relu(sf2_ref[...] + os_ref[...]))
    h1 = jax.nn.relu(
        jnp.dot(r, f1a_ref[...], preferred_element_type=jnp.float32)
        + jnp.dot(g, f1b_ref[...], preferred_element_type=jnp.float32)
        + f1bias_ref[...]
    )
    h = jnp.dot(h1, f2w_ref[...], preferred_element_type=jnp.float32) + f2bias_ref[...]
    out_ref[...] = _normalize_rows(z_ref[...] + jax.nn.relu(h))


def _post(z, self_f, out_t, self_f2, out_s, F1A, F1B, F1bias, F2W, F2bias):
    grid = (N // ROWS_BLK,)
    row_spec = lambda c: pl.BlockSpec((ROWS_BLK, c), lambda i: (i, 0))
    full_spec = lambda r, c: pl.BlockSpec((r, c), lambda i: (0, 0))
    return pl.pallas_call(
        _post_body,
        grid=grid,
        in_specs=[
            row_spec(DIN), row_spec(DH), row_spec(DH), row_spec(DH), row_spec(DH),
            full_spec(DH, DIN), full_spec(DH, DIN), full_spec(1, DIN),
            full_spec(DIN, DIN), full_spec(1, DIN),
        ],
        out_specs=pl.BlockSpec((ROWS_BLK, DIN), lambda i: (i, 0)),
        out_shape=jax.ShapeDtypeStruct((N, DIN), jnp.float32),
    )(z, self_f, out_t, self_f2, out_s, F1A, F1B, F1bias, F2W, F2bias)


EPT_A = EU // NS
NCHUNK_A = EPT_A // K


@functools.partial(
    pl.kernel,
    out_type=jax.ShapeDtypeStruct((NC, NPAD, HALF), jnp.float32),
    mesh=_MESH,
    compiler_params=_SC_PARAMS,
    scratch_types=[
        pltpu.VMEM_SHARED((NPAD, HALF), jnp.float32),
        pltpu.VMEM((3, K), jnp.int32),
        pltpu.VMEM((3, K), jnp.int32),
        pltpu.VMEM((2, K, HALF), jnp.float32),
        pltpu.VMEM((2, K, HALF), jnp.float32),
        pltpu.SemaphoreType.DMA((3,)),
        pltpu.SemaphoreType.DMA((2,)),
        pltpu.SemaphoreType.DMA((2,)),
        pltpu.SemaphoreType.DMA((2,)),
        pltpu.SemaphoreType.DMA((2,)),
    ],
)
def _sc_temporal(src_hbm, dst_hbm, zwt_hbm, tf_hbm, zeros_hbm, out_hbm,
                 acc, sbuf, dbuf, rows, tfb, isem, gsem, tsem, ssem, tssem):
    c = lax.axis_index("c")
    s = lax.axis_index("s")

    pltpu.sync_copy(zeros_hbm, acc.at[pl.ds(s * NPT, NPT)])
    plsc.subcore_barrier()

    def idx_descs(k, b3):
        base = s * EPT_A + k * K
        return (
            pltpu.make_async_copy(src_hbm.at[pl.ds(base, K)], sbuf.at[b3],
                                  isem.at[b3]),
            pltpu.make_async_copy(dst_hbm.at[pl.ds(base, K)], dbuf.at[b3],
                                  isem.at[b3]),
        )

    def issue_idx(k):
        for d in idx_descs(k, lax.rem(k, 3)):
            d.start()

    def wait_idx(k):
        for d in idx_descs(k, lax.rem(k, 3)):
            d.wait()

    def gather_desc(k):
        b2 = lax.rem(k, 2)
        b3 = lax.rem(k, 3)
        return pltpu.make_async_copy(
            zwt_hbm.at[c].at[sbuf.at[b3]], rows.at[b2], gsem.at[b2])

    def tf_desc(k):
        b2 = lax.rem(k, 2)
        base = s * EPT_A + k * K
        return pltpu.make_async_copy(
            tf_hbm.at[pl.ds(base, K), pl.ds(c * HALF, HALF)],
            tfb.at[b2], tsem.at[b2])

    def scat_desc(k):
        b2 = lax.rem(k, 2)
        b3 = lax.rem(k, 3)
        return pltpu.make_async_copy(
            rows.at[b2], acc.at[dbuf.at[b3]], ssem.at[b2])

    def scat_tf_desc(k):
        b2 = lax.rem(k, 2)
        b3 = lax.rem(k, 3)
        return pltpu.make_async_copy(
            tfb.at[b2], acc.at[dbuf.at[b3]], tssem.at[b2])

    issue_idx(0)
    wait_idx(0)
    gather_desc(0).start()
    tf_desc(0).start()
    issue_idx(1)

    def chunk(k, carry):
        b2 = lax.rem(k, 2)

        @pl.when(k >= 1)
        def _drain_scatter():
            scat_desc(k - 1).wait()
            scat_tf_desc(k - 1).wait()

        @pl.when(k + 1 < NCHUNK_A)
        def _next_streams():
            wait_idx(k + 1)
            gather_desc(k + 1).start()
            tf_desc(k + 1).start()

        @pl.when(k + 2 < NCHUNK_A)
        def _next_idx():
            issue_idx(k + 2)

        gather_desc(k).wait()
        scat_desc(k).start(add=True)
        tf_desc(k).wait()
        scat_tf_desc(k).start(add=True)
        return carry

    lax.fori_loop(0, NCHUNK_A, chunk, 0)
    scat_desc(NCHUNK_A - 1).wait()
    scat_tf_desc(NCHUNK_A - 1).wait()
    plsc.subcore_barrier()
    pltpu.sync_copy(acc.at[pl.ds(s * NPT, NPT)],
                    out_hbm.at[c].at[pl.ds(s * NPT, NPT)])


EPT_B = E // NS
NCHUNK_B = EPT_B // K


@functools.partial(
    pl.kernel,
    out_type=jax.ShapeDtypeStruct((NC, NPAD, HALF), jnp.float32),
    mesh=_MESH,
    compiler_params=_SC_PARAMS,
    scratch_types=[
        pltpu.VMEM_SHARED((NPAD, HALF), jnp.float32),
        pltpu.VMEM_SHARED((NPAD,), jnp.float32),
        pltpu.VMEM((NPAD,), jnp.float32),
        pltpu.VMEM((3, K), jnp.int32),
        pltpu.VMEM((3, K), jnp.int32),
        pltpu.VMEM((3, K), jnp.float32),
        pltpu.VMEM((2, K, HALF), jnp.float32),
        pltpu.SemaphoreType.DMA((3,)),
        pltpu.SemaphoreType.DMA((2,)),
        pltpu.SemaphoreType.DMA,
        pltpu.SemaphoreType.DMA((2,)),
    ],
)
def _sc_structural(row_hbm, col_hbm, td_hbm, zwn_hbm, zeros_hbm, zcol_hbm,
                   out_hbm, acc, tds, tdl, rbuf, cbuf, tbuf, rows,
                   isem, gsem, psem, ssem):
    c = lax.axis_index("c")
    s = lax.axis_index("s")

    pltpu.sync_copy(zeros_hbm, acc.at[pl.ds(s * NPT, NPT)])
    pltpu.sync_copy(zcol_hbm, tds.at[pl.ds(s * NPT, NPT)])
    plsc.subcore_barrier()

    def idx1_descs(k, b3):
        base = s * EPT_B + k * K
        return (
            pltpu.make_async_copy(row_hbm.at[pl.ds(base, K)], rbuf.at[b3],
                                  isem.at[b3]),
            pltpu.make_async_copy(td_hbm.at[pl.ds(base, K)], tbuf.at[b3],
                                  isem.at[b3]),
        )

    def issue_idx1(k):
        for d in idx1_descs(k, lax.rem(k, 3)):
            d.start()

    def wait_idx1(k):
        for d in idx1_descs(k, lax.rem(k, 3)):
            d.wait()

    def p1scat_desc(k):
        b3 = lax.rem(k, 3)
        return pltpu.make_async_copy(tbuf.at[b3], tds.at[rbuf.at[b3]], psem)

    issue_idx1(0)
    issue_idx1(1)

    def p1(k, carry):
        wait_idx1(k)
        p1scat_desc(k).start(add=True)

        @pl.when(k >= 1)
        def _drain_prev():
            p1scat_desc(k - 1).wait()

        @pl.when(k + 2 < NCHUNK_B)
        def _next_idx():
            issue_idx1(k + 2)

        return carry

    lax.fori_loop(0, NCHUNK_B, p1, 0)
    p1scat_desc(NCHUNK_B - 1).wait()
    plsc.subcore_barrier()

    pltpu.sync_copy(tds, tdl)

    def fix(i, carry):
        sl = pl.ds(i * LANES, LANES)
        v = tdl[sl]
        tdl[sl] = jnp.where(v == 0.0, 1.0, v)
        return carry

    lax.fori_loop(0, NPAD // LANES, fix, 0)

    def idx3_descs(k, b3):
        base = s * EPT_B + k * K
        return (
            pltpu.make_async_copy(row_hbm.at[pl.ds(base, K)], rbuf.at[b3],
                                  isem.at[b3]),
            pltpu.make_async_copy(col_hbm.at[pl.ds(base, K)], cbuf.at[b3],
                                  isem.at[b3]),
            pltpu.make_async_copy(td_hbm.at[pl.ds(base, K)], tbuf.at[b3],
                                  isem.at[b3]),
        )

    def issue_idx3(k):
        for d in idx3_descs(k, lax.rem(k, 3)):
            d.start()

    def wait_idx3(k):
        for d in idx3_descs(k, lax.rem(k, 3)):
            d.wait()

    def gather_desc(k):
        b2 = lax.rem(k, 2)
        b3 = lax.rem(k, 3)
        return pltpu.make_async_copy(
            zwn_hbm.at[c].at[rbuf.at[b3]], rows.at[b2], gsem.at[b2])

    def scat_desc(k):
        b2 = lax.rem(k, 2)
        b3 = lax.rem(k, 3)
        return pltpu.make_async_copy(
            rows.at[b2], acc.at[cbuf.at[b3]], ssem.at[b2])

    issue_idx3(0)
    wait_idx3(0)
    gather_desc(0).start()
    issue_idx3(1)

    def chunk3(k, carry):
        b3 = lax.rem(k, 3)

        @pl.when(k >= 1)
        def _drain_scatter():
            scat_desc(k - 1).wait()

        @pl.when(k + 1 < NCHUNK_B)
        def _next_gather():
            wait_idx3(k + 1)
            gather_desc(k + 1).start()

        @pl.when(k + 2 < NCHUNK_B)
        def _next_idx():
            issue_idx3(k + 2)

        @plsc.parallel_loop(0, K // LANES, unroll=5)
        def relw(j):
            sl = pl.ds(j * LANES, LANES)
            den = plsc.load_gather(tdl, [rbuf[b3, sl]])
            tbuf[b3, sl] = tbuf[b3, sl] / den
        gather_desc(k).wait()
        b2 = lax.rem(k, 2)

        @plsc.parallel_loop(0, K, unroll=4)
        def scale(e):
            w = plsc.load_gather(
                tbuf, [jnp.full((LANES,), b3, jnp.int32),
                       jnp.full((LANES,), e, jnp.int32)])
            for j in range(JV):
                sl = pl.ds(j * LANES, LANES)
                rows[b2, e, sl] = rows[b2, e, sl] * w
        scat_desc(k).start(add=True)
        return carry

    lax.fori_loop(0, NCHUNK_B, chunk3, 0)
    scat_desc(NCHUNK_B - 1).wait()
    plsc.subcore_barrier()
    pltpu.sync_copy(acc.at[pl.ds(s * NPT, NPT)],
                    out_hbm.at[c].at[pl.ds(s * NPT, NPT)])


def kernel(z, edge_index, temporal_features, time_diffs, unique_edges,
           WS1, bS1, WT, bT, WS2, bS2, WN, bN, F1W, F1b, F2W, F2b):
    wcat = jnp.concatenate([WS1, WT, WS2, WN], axis=1)
    bcat = jnp.concatenate([bS1, bT, bS2, bN]).reshape(1, 4 * DH)
    pre = _pre_matmul(z, wcat, bcat)
    self_f = pre[:, 0:DH]
    self_f2 = pre[:, 2 * DH:3 * DH]
    zwt2 = jnp.stack([pre[:, DH:DH + HALF], pre[:, DH + HALF:2 * DH]])
    zwn2 = jnp.stack([pre[:, 3 * DH:3 * DH + HALF], pre[:, 3 * DH + HALF:4 * DH]])

    zeros = jnp.zeros((NPT, HALF), jnp.float32)
    zcol = jnp.zeros((NPT,), jnp.float32)

    out_t2 = _sc_temporal(unique_edges[0], unique_edges[1], zwt2,
                          temporal_features, zeros)
    out_t = out_t2[:, :N, :].transpose(1, 0, 2).reshape(N, DH)

    td = time_diffs.astype(jnp.float32)
    out_s2 = _sc_structural(edge_index[0], edge_index[1], td, zwn2,
                            zeros, zcol)
    out_s = out_s2[:, :N, :].transpose(1, 0, 2).reshape(N, DH)

    F1A = F1W[:DH]
    F1B = F1W[DH:]
    return _post(z, self_f, out_t, self_f2, out_s,
                 F1A, F1B, F1b.reshape(1, DIN), F2W, F2b.reshape(1, DIN))


# speedup vs baseline: 1.7881x; 1.0659x over previous
import functools

import jax
import jax.numpy as jnp
from jax import lax
from jax.experimental import pallas as pl
from jax.experimental.pallas import tpu as pltpu
from jax.experimental.pallas import tpu_sc as plsc

N = 10000
E = 320000
EU = 160000
DIN = 128
DH = 256

NC = 2
NS = 16
LANES = 16

ROWS_BLK = 1000
HALF = DH // NC
NPAD = 10240
NPT = NPAD // NS
K = 80
JV = HALF // LANES

_MESH = plsc.VectorSubcoreMesh(core_axis_name="c", subcore_axis_name="s",
                               num_cores=NC, num_subcores=NS)
_SC_PARAMS = pltpu.CompilerParams(needs_layout_passes=False)


def _pre_matmul_body(z_ref, w_ref, b_ref, selfs_ref, gt_ref):
    x = (jnp.dot(z_ref[...], w_ref[...], preferred_element_type=jnp.float32)
         + b_ref[...])
    selfs_ref[...] = x[:, :2 * DH]
    gt_ref[0, 0] = x[:, 2 * DH:2 * DH + HALF]
    gt_ref[0, 1] = x[:, 2 * DH + HALF:3 * DH]
    gt_ref[1, 0] = x[:, 3 * DH:3 * DH + HALF]
    gt_ref[1, 1] = x[:, 3 * DH + HALF:4 * DH]


def _pre_matmul(z, wcat, bcat):
    grid = (N // ROWS_BLK,)
    return pl.pallas_call(
        _pre_matmul_body,
        grid=grid,
        in_specs=[
            pl.BlockSpec((ROWS_BLK, DIN), lambda i: (i, 0)),
            pl.BlockSpec((DIN, 4 * DH), lambda i: (0, 0)),
            pl.BlockSpec((1, 4 * DH), lambda i: (0, 0)),
        ],
        out_specs=[
            pl.BlockSpec((ROWS_BLK, 2 * DH), lambda i: (i, 0)),
            pl.BlockSpec((2, 2, ROWS_BLK, HALF), lambda i: (0, 0, i, 0)),
        ],
        out_shape=[
            jax.ShapeDtypeStruct((N, 2 * DH), jnp.float32),
            jax.ShapeDtypeStruct((2, 2, N, HALF), jnp.float32),
        ],
    )(z, wcat, bcat)


def _normalize_rows(x):
    n = jnp.sqrt(jnp.sum(x * x, axis=1, keepdims=True))
    return x / jnp.maximum(n, 1e-12)


def _post_body(z_ref, selfs_ref, ot_ref, os_ref,
               f1a_ref, f1b_ref, f1bias_ref, f2w_ref, f2bias_ref, out_ref):
    sf = selfs_ref[:, :DH]
    sf2 = selfs_ref[:, DH:]
    ot = jnp.concatenate([ot_ref[0], ot_ref[1]], axis=-1)
    os_ = jnp.concatenate([os_ref[0], os_ref[1]], axis=-1)
    r = _normalize_rows(jax.nn.relu(sf + ot))
    g = _normalize_rows(jax.nn.relu(sf2 + os_))
    h1 = jax.nn.relu(
        jnp.dot(r, f1a_ref[...], preferred_element_type=jnp.float32)
        + jnp.dot(g, f1b_ref[...], preferred_element_type=jnp.float32)
        + f1bias_ref[...]
    )
    h = jnp.dot(h1, f2w_ref[...], preferred_element_type=jnp.float32) + f2bias_ref[...]
    out_ref[...] = _normalize_rows(z_ref[...] + jax.nn.relu(h))


def _post(z, selfs, out_t2, out_s2, F1A, F1B, F1bias, F2W, F2bias):
    grid = (N // ROWS_BLK,)
    full_spec = lambda r, c: pl.BlockSpec((r, c), lambda i: (0, 0))
    return pl.pallas_call(
        _post_body,
        grid=grid,
        in_specs=[
            pl.BlockSpec((ROWS_BLK, DIN), lambda i: (i, 0)),
            pl.BlockSpec((ROWS_BLK, 2 * DH), lambda i: (i, 0)),
            pl.BlockSpec((2, ROWS_BLK, HALF), lambda i: (0, i, 0)),
            pl.BlockSpec((2, ROWS_BLK, HALF), lambda i: (0, i, 0)),
            full_spec(DH, DIN), full_spec(DH, DIN), full_spec(1, DIN),
            full_spec(DIN, DIN), full_spec(1, DIN),
        ],
        out_specs=pl.BlockSpec((ROWS_BLK, DIN), lambda i: (i, 0)),
        out_shape=jax.ShapeDtypeStruct((N, DIN), jnp.float32),
    )(z, selfs, out_t2, out_s2, F1A, F1B, F1bias, F2W, F2bias)


EPT_A = EU // NS
NCHUNK_A = EPT_A // K


@functools.partial(
    pl.kernel,
    out_type=jax.ShapeDtypeStruct((NC, NPAD, HALF), jnp.float32),
    mesh=_MESH,
    compiler_params=_SC_PARAMS,
    scratch_types=[
        pltpu.VMEM_SHARED((NPAD, HALF), jnp.float32),
        pltpu.VMEM((3, K), jnp.int32),
        pltpu.VMEM((3, K), jnp.int32),
        pltpu.VMEM((2, K, HALF), jnp.float32),
        pltpu.VMEM((2, K, HALF), jnp.float32),
        pltpu.SemaphoreType.DMA((3,)),
        pltpu.SemaphoreType.DMA((2,)),
        pltpu.SemaphoreType.DMA((2,)),
        pltpu.SemaphoreType.DMA((2,)),
        pltpu.SemaphoreType.DMA((2,)),
    ],
)
def _sc_temporal(src_hbm, dst_hbm, zwt_hbm, tf_hbm, zeros_hbm, out_hbm,
                 acc, sbuf, dbuf, rows, tfb, isem, gsem, tsem, ssem, tssem):
    c = lax.axis_index("c")
    s = lax.axis_index("s")

    pltpu.sync_copy(zeros_hbm, acc.at[pl.ds(s * NPT, NPT)])
    plsc.subcore_barrier()

    def idx_descs(k, b3):
        base = s * EPT_A + k * K
        return (
            pltpu.make_async_copy(src_hbm.at[pl.ds(base, K)], sbuf.at[b3],
                                  isem.at[b3]),
            pltpu.make_async_copy(dst_hbm.at[pl.ds(base, K)], dbuf.at[b3],
                                  isem.at[b3]),
        )

    def issue_idx(k):
        for d in idx_descs(k, lax.rem(k, 3)):
            d.start()

    def wait_idx(k):
        for d in idx_descs(k, lax.rem(k, 3)):
            d.wait()

    def gather_desc(k):
        b2 = lax.rem(k, 2)
        b3 = lax.rem(k, 3)
        return pltpu.make_async_copy(
            zwt_hbm.at[0].at[c].at[sbuf.at[b3]], rows.at[b2], gsem.at[b2])

    def tf_desc(k):
        b2 = lax.rem(k, 2)
        base = s * EPT_A + k * K
        return pltpu.make_async_copy(
            tf_hbm.at[pl.ds(base, K), pl.ds(c * HALF, HALF)],
            tfb.at[b2], tsem.at[b2])

    def scat_desc(k):
        b2 = lax.rem(k, 2)
        b3 = lax.rem(k, 3)
        return pltpu.make_async_copy(
            rows.at[b2], acc.at[dbuf.at[b3]], ssem.at[b2])

    def scat_tf_desc(k):
        b2 = lax.rem(k, 2)
        b3 = lax.rem(k, 3)
        return pltpu.make_async_copy(
            tfb.at[b2], acc.at[dbuf.at[b3]], tssem.at[b2])

    issue_idx(0)
    wait_idx(0)
    gather_desc(0).start()
    tf_desc(0).start()
    issue_idx(1)

    def chunk(k, carry):
        b2 = lax.rem(k, 2)

        @pl.when(k >= 1)
        def _drain_scatter():
            scat_desc(k - 1).wait()
            scat_tf_desc(k - 1).wait()

        @pl.when(k + 1 < NCHUNK_A)
        def _next_streams():
            wait_idx(k + 1)
            gather_desc(k + 1).start()
            tf_desc(k + 1).start()

        @pl.when(k + 2 < NCHUNK_A)
        def _next_idx():
            issue_idx(k + 2)

        gather_desc(k).wait()
        scat_desc(k).start(add=True)
        tf_desc(k).wait()
        scat_tf_desc(k).start(add=True)
        return carry

    lax.fori_loop(0, NCHUNK_A, chunk, 0)
    scat_desc(NCHUNK_A - 1).wait()
    scat_tf_desc(NCHUNK_A - 1).wait()
    plsc.subcore_barrier()
    pltpu.sync_copy(acc.at[pl.ds(s * NPT, NPT)],
                    out_hbm.at[c].at[pl.ds(s * NPT, NPT)])


EPT_B = E // NS
NCHUNK_B = EPT_B // K


@functools.partial(
    pl.kernel,
    out_type=jax.ShapeDtypeStruct((NC, NPAD, HALF), jnp.float32),
    mesh=_MESH,
    compiler_params=_SC_PARAMS,
    scratch_types=[
        pltpu.VMEM_SHARED((NPAD, HALF), jnp.float32),
        pltpu.VMEM_SHARED((NPAD,), jnp.float32),
        pltpu.VMEM((NPAD,), jnp.float32),
        pltpu.VMEM((3, K), jnp.int32),
        pltpu.VMEM((3, K), jnp.int32),
        pltpu.VMEM((3, K), jnp.float32),
        pltpu.VMEM((2, K, HALF), jnp.float32),
        pltpu.SemaphoreType.DMA((3,)),
        pltpu.SemaphoreType.DMA((2,)),
        pltpu.SemaphoreType.DMA,
        pltpu.SemaphoreType.DMA((2,)),
    ],
)
def _sc_structural(row_hbm, col_hbm, td_hbm, zwn_hbm, zeros_hbm, zcol_hbm,
                   out_hbm, acc, tds, tdl, rbuf, cbuf, tbuf, rows,
                   isem, gsem, psem, ssem):
    c = lax.axis_index("c")
    s = lax.axis_index("s")

    pltpu.sync_copy(zeros_hbm, acc.at[pl.ds(s * NPT, NPT)])
    pltpu.sync_copy(zcol_hbm, tds.at[pl.ds(s * NPT, NPT)])
    plsc.subcore_barrier()

    def idx1_descs(k, b3):
        base = s * EPT_B + k * K
        return (
            pltpu.make_async_copy(row_hbm.at[pl.ds(base, K)], rbuf.at[b3],
                                  isem.at[b3]),
            pltpu.make_async_copy(td_hbm.at[pl.ds(base, K)], tbuf.at[b3],
                                  isem.at[b3]),
        )

    def issue_idx1(k):
        for d in idx1_descs(k, lax.rem(k, 3)):
            d.start()

    def wait_idx1(k):
        for d in idx1_descs(k, lax.rem(k, 3)):
            d.wait()

    def p1scat_desc(k):
        b3 = lax.rem(k, 3)
        return pltpu.make_async_copy(tbuf.at[b3], tds.at[rbuf.at[b3]], psem)

    issue_idx1(0)
    issue_idx1(1)

    def p1(k, carry):
        wait_idx1(k)
        p1scat_desc(k).start(add=True)

        @pl.when(k >= 1)
        def _drain_prev():
            p1scat_desc(k - 1).wait()

        @pl.when(k + 2 < NCHUNK_B)
        def _next_idx():
            issue_idx1(k + 2)

        return carry

    lax.fori_loop(0, NCHUNK_B, p1, 0)
    p1scat_desc(NCHUNK_B - 1).wait()
    plsc.subcore_barrier()

    pltpu.sync_copy(tds, tdl)

    def fix(i, carry):
        sl = pl.ds(i * LANES, LANES)
        v = tdl[sl]
        tdl[sl] = jnp.where(v == 0.0, 1.0, v)
        return carry

    lax.fori_loop(0, NPAD // LANES, fix, 0)

    def idx3_descs(k, b3):
        base = s * EPT_B + k * K
        return (
            pltpu.make_async_copy(row_hbm.at[pl.ds(base, K)], rbuf.at[b3],
                                  isem.at[b3]),
            pltpu.make_async_copy(col_hbm.at[pl.ds(base, K)], cbuf.at[b3],
                                  isem.at[b3]),
            pltpu.make_async_copy(td_hbm.at[pl.ds(base, K)], tbuf.at[b3],
                                  isem.at[b3]),
        )

    def issue_idx3(k):
        for d in idx3_descs(k, lax.rem(k, 3)):
            d.start()

    def wait_idx3(k):
        for d in idx3_descs(k, lax.rem(k, 3)):
            d.wait()

    def gather_desc(k):
        b2 = lax.rem(k, 2)
        b3 = lax.rem(k, 3)
        return pltpu.make_async_copy(
            zwn_hbm.at[1].at[c].at[rbuf.at[b3]], rows.at[b2], gsem.at[b2])

    def scat_desc(k):
        b2 = lax.rem(k, 2)
        b3 = lax.rem(k, 3)
        return pltpu.make_async_copy(
            rows.at[b2], acc.at[cbuf.at[b3]], ssem.at[b2])

    issue_idx3(0)
    wait_idx3(0)
    gather_desc(0).start()
    issue_idx3(1)

    def chunk3(k, carry):
        b3 = lax.rem(k, 3)

        @pl.when(k >= 1)
        def _drain_scatter():
            scat_desc(k - 1).wait()

        @pl.when(k + 1 < NCHUNK_B)
        def _next_gather():
            wait_idx3(k + 1)
            gather_desc(k + 1).start()

        @pl.when(k + 2 < NCHUNK_B)
        def _next_idx():
            issue_idx3(k + 2)

        @plsc.parallel_loop(0, K // LANES, unroll=5)
        def relw(j):
            sl = pl.ds(j * LANES, LANES)
            den = plsc.load_gather(tdl, [rbuf[b3, sl]])
            tbuf[b3, sl] = tbuf[b3, sl] / den
        gather_desc(k).wait()
        b2 = lax.rem(k, 2)

        @plsc.parallel_loop(0, K, unroll=4)
        def scale(e):
            w = plsc.load_gather(
                tbuf, [jnp.full((LANES,), b3, jnp.int32),
                       jnp.full((LANES,), e, jnp.int32)])
            for j in range(JV):
                sl = pl.ds(j * LANES, LANES)
                rows[b2, e, sl] = rows[b2, e, sl] * w
        scat_desc(k).start(add=True)
        return carry

    lax.fori_loop(0, NCHUNK_B, chunk3, 0)
    scat_desc(NCHUNK_B - 1).wait()
    plsc.subcore_barrier()
    pltpu.sync_copy(acc.at[pl.ds(s * NPT, NPT)],
                    out_hbm.at[c].at[pl.ds(s * NPT, NPT)])


def kernel(z, edge_index, temporal_features, time_diffs, unique_edges,
           WS1, bS1, WT, bT, WS2, bS2, WN, bN, F1W, F1b, F2W, F2b):
    wcat = jnp.concatenate([WS1, WS2, WT, WN], axis=1)
    bcat = jnp.concatenate([bS1, bS2, bT, bN]).reshape(1, 4 * DH)
    selfs, gt = _pre_matmul(z, wcat, bcat)

    zeros = jnp.zeros((NPT, HALF), jnp.float32)
    zcol = jnp.zeros((NPT,), jnp.float32)

    out_t2 = _sc_temporal(unique_edges[0], unique_edges[1], gt,
                          temporal_features, zeros)

    td = time_diffs.astype(jnp.float32)
    out_s2 = _sc_structural(edge_index[0], edge_index[1], td, gt,
                            zeros, zcol)

    F1A = F1W[:DH]
    F1B = F1W[DH:]
    return _post(z, selfs, out_t2, out_s2,
                 F1A, F1B, F1b.reshape(1, DIN), F2W, F2b.reshape(1, DIN))


# speedup vs baseline: 1.7897x; 1.0009x over previous
import functools

import jax
import jax.numpy as jnp
from jax import lax
from jax.experimental import pallas as pl
from jax.experimental.pallas import tpu as pltpu
from jax.experimental.pallas import tpu_sc as plsc

N = 10000
E = 320000
EU = 160000
DIN = 128
DH = 256

NC = 2
NS = 16
LANES = 16

ROWS_BLK = 1000
HALF = DH // NC
NPAD = 10240
NPT = NPAD // NS
K = 80
JV = HALF // LANES

_MESH = plsc.VectorSubcoreMesh(core_axis_name="c", subcore_axis_name="s",
                               num_cores=NC, num_subcores=NS)
_SC_PARAMS = pltpu.CompilerParams(needs_layout_passes=False)


def _pre_matmul_body(z_ref, w_ref, b_ref, selfs_ref, gt_ref):
    x = (jnp.dot(z_ref[...], w_ref[...], preferred_element_type=jnp.float32)
         + b_ref[...])
    selfs_ref[...] = x[:, :2 * DH]
    gt_ref[0, 0] = x[:, 2 * DH:2 * DH + HALF]
    gt_ref[0, 1] = x[:, 2 * DH + HALF:3 * DH]
    gt_ref[1, 0] = x[:, 3 * DH:3 * DH + HALF]
    gt_ref[1, 1] = x[:, 3 * DH + HALF:4 * DH]


def _pre_matmul(z, wcat, bcat):
    grid = (N // ROWS_BLK,)
    return pl.pallas_call(
        _pre_matmul_body,
        grid=grid,
        in_specs=[
            pl.BlockSpec((ROWS_BLK, DIN), lambda i: (i, 0)),
            pl.BlockSpec((DIN, 4 * DH), lambda i: (0, 0)),
            pl.BlockSpec((1, 4 * DH), lambda i: (0, 0)),
        ],
        out_specs=[
            pl.BlockSpec((ROWS_BLK, 2 * DH), lambda i: (i, 0)),
            pl.BlockSpec((2, 2, ROWS_BLK, HALF), lambda i: (0, 0, i, 0)),
        ],
        out_shape=[
            jax.ShapeDtypeStruct((N, 2 * DH), jnp.float32),
            jax.ShapeDtypeStruct((2, 2, N, HALF), jnp.float32),
        ],
    )(z, wcat, bcat)


def _normalize_rows(x):
    n = jnp.sqrt(jnp.sum(x * x, axis=1, keepdims=True))
    return x / jnp.maximum(n, 1e-12)


def _post_body(z_ref, selfs_ref, ot_ref, os_ref,
               f1a_ref, f1b_ref, f1bias_ref, f2w_ref, f2bias_ref, out_ref):
    sf = selfs_ref[:, :DH]
    sf2 = selfs_ref[:, DH:]
    ot = jnp.concatenate([ot_ref[0], ot_ref[1]], axis=-1)
    os_ = jnp.concatenate([os_ref[0], os_ref[1]], axis=-1)
    r = _normalize_rows(jax.nn.relu(sf + ot))
    g = _normalize_rows(jax.nn.relu(sf2 + os_))
    h1 = jax.nn.relu(
        jnp.dot(r, f1a_ref[...], preferred_element_type=jnp.float32)
        + jnp.dot(g, f1b_ref[...], preferred_element_type=jnp.float32)
        + f1bias_ref[...]
    )
    h = jnp.dot(h1, f2w_ref[...], preferred_element_type=jnp.float32) + f2bias_ref[...]
    out_ref[...] = _normalize_rows(z_ref[...] + jax.nn.relu(h))


def _post(z, selfs, out_t2, out_s2, F1A, F1B, F1bias, F2W, F2bias):
    grid = (N // ROWS_BLK,)
    full_spec = lambda r, c: pl.BlockSpec((r, c), lambda i: (0, 0))
    return pl.pallas_call(
        _post_body,
        grid=grid,
        in_specs=[
            pl.BlockSpec((ROWS_BLK, DIN), lambda i: (i, 0)),
            pl.BlockSpec((ROWS_BLK, 2 * DH), lambda i: (i, 0)),
            pl.BlockSpec((2, ROWS_BLK, HALF), lambda i: (0, i, 0)),
            pl.BlockSpec((2, ROWS_BLK, HALF), lambda i: (0, i, 0)),
            full_spec(DH, DIN), full_spec(DH, DIN), full_spec(1, DIN),
            full_spec(DIN, DIN), full_spec(1, DIN),
        ],
        out_specs=pl.BlockSpec((ROWS_BLK, DIN), lambda i: (i, 0)),
        out_shape=jax.ShapeDtypeStruct((N, DIN), jnp.float32),
    )(z, selfs, out_t2, out_s2, F1A, F1B, F1bias, F2W, F2bias)


EPT_A = EU // NS
NCHUNK_A = EPT_A // K


@functools.partial(
    pl.kernel,
    out_type=jax.ShapeDtypeStruct((NC, NPAD, HALF), jnp.float32),
    mesh=_MESH,
    compiler_params=_SC_PARAMS,
    scratch_types=[
        pltpu.VMEM_SHARED((NPAD, HALF), jnp.float32),
        pltpu.VMEM((3, K), jnp.int32),
        pltpu.VMEM((3, K), jnp.int32),
        pltpu.VMEM((2, K, HALF), jnp.float32),
        pltpu.VMEM((2, K, HALF), jnp.float32),
        pltpu.SemaphoreType.DMA((3,)),
        pltpu.SemaphoreType.DMA((2,)),
        pltpu.SemaphoreType.DMA((2,)),
        pltpu.SemaphoreType.DMA((2,)),
        pltpu.SemaphoreType.DMA((2,)),
    ],
)
def _sc_temporal(src_hbm, dst_hbm, zwt_hbm, tf_hbm, zeros_hbm, out_hbm,
                 acc, sbuf, dbuf, rows, tfb, isem, gsem, tsem, ssem, tssem):
    c = lax.axis_index("c")
    s = lax.axis_index("s")

    pltpu.sync_copy(zeros_hbm, acc.at[pl.ds(s * NPT, NPT)])
    plsc.subcore_barrier()

    def idx_descs(k, b3):
        base = s * EPT_A + k * K
        return (
            pltpu.make_async_copy(src_hbm.at[pl.ds(base, K)], sbuf.at[b3],
                                  isem.at[b3]),
            pltpu.make_async_copy(dst_hbm.at[pl.ds(base, K)], dbuf.at[b3],
                                  isem.at[b3]),
        )

    def issue_idx(k):
        for d in idx_descs(k, lax.rem(k, 3)):
            d.start()

    def wait_idx(k):
        for d in idx_descs(k, lax.rem(k, 3)):
            d.wait()

    def gather_desc(k):
        b2 = lax.rem(k, 2)
        b3 = lax.rem(k, 3)
        return pltpu.make_async_copy(
            zwt_hbm.at[0].at[c].at[sbuf.at[b3]], rows.at[b2], gsem.at[b2])

    def tf_desc(k):
        b2 = lax.rem(k, 2)
        base = s * EPT_A + k * K
        return pltpu.make_async_copy(
            tf_hbm.at[pl.ds(base, K), pl.ds(c * HALF, HALF)],
            tfb.at[b2], tsem.at[b2])

    def scat_desc(k):
        b2 = lax.rem(k, 2)
        b3 = lax.rem(k, 3)
        return pltpu.make_async_copy(
            rows.at[b2], acc.at[dbuf.at[b3]], ssem.at[b2])

    def scat_tf_desc(k):
        b2 = lax.rem(k, 2)
        b3 = lax.rem(k, 3)
        return pltpu.make_async_copy(
            tfb.at[b2], acc.at[dbuf.at[b3]], tssem.at[b2])

    issue_idx(0)
    wait_idx(0)
    gather_desc(0).start()
    tf_desc(0).start()
    issue_idx(1)

    def chunk(k, carry):
        b2 = lax.rem(k, 2)

        @pl.when(k >= 1)
        def _drain_scatter():
            scat_desc(k - 1).wait()
            scat_tf_desc(k - 1).wait()

        @pl.when(k + 1 < NCHUNK_A)
        def _next_streams():
            wait_idx(k + 1)
            gather_desc(k + 1).start()
            tf_desc(k + 1).start()

        @pl.when(k + 2 < NCHUNK_A)
        def _next_idx():
            issue_idx(k + 2)

        gather_desc(k).wait()
        scat_desc(k).start(add=True)
        tf_desc(k).wait()
        scat_tf_desc(k).start(add=True)
        return carry

    lax.fori_loop(0, NCHUNK_A, chunk, 0)
    scat_desc(NCHUNK_A - 1).wait()
    scat_tf_desc(NCHUNK_A - 1).wait()
    plsc.subcore_barrier()
    pltpu.sync_copy(acc.at[pl.ds(s * NPT, NPT)],
                    out_hbm.at[c].at[pl.ds(s * NPT, NPT)])


EPT_B = E // NS
NCHUNK_B = EPT_B // K


@functools.partial(
    pl.kernel,
    out_type=jax.ShapeDtypeStruct((NC, NPAD, HALF), jnp.float32),
    mesh=_MESH,
    compiler_params=_SC_PARAMS,
    scratch_types=[
        pltpu.VMEM_SHARED((NPAD, HALF), jnp.float32),
        pltpu.VMEM_SHARED((NPAD,), jnp.float32),
        pltpu.VMEM((NPAD,), jnp.float32),
        pltpu.VMEM((3, K), jnp.int32),
        pltpu.VMEM((3, K), jnp.int32),
        pltpu.VMEM((3, K), jnp.float32),
        pltpu.VMEM((2, K, HALF), jnp.float32),
        pltpu.SemaphoreType.DMA((3,)),
        pltpu.SemaphoreType.DMA((2,)),
        pltpu.SemaphoreType.DMA,
        pltpu.SemaphoreType.DMA((2,)),
    ],
)
def _sc_structural(row_hbm, col_hbm, td_hbm, zwn_hbm, zeros_hbm, zcol_hbm,
                   out_hbm, acc, tds, tdl, rbuf, cbuf, tbuf, rows,
                   isem, gsem, psem, ssem):
    c = lax.axis_index("c")
    s = lax.axis_index("s")

    pltpu.sync_copy(zeros_hbm, acc.at[pl.ds(s * NPT, NPT)])
    pltpu.sync_copy(zcol_hbm, tds.at[pl.ds(s * NPT, NPT)])
    plsc.subcore_barrier()

    def idx1_descs(k, b3):
        base = s * EPT_B + k * K
        return (
            pltpu.make_async_copy(row_hbm.at[pl.ds(base, K)], rbuf.at[b3],
                                  isem.at[b3]),
            pltpu.make_async_copy(td_hbm.at[pl.ds(base, K)], tbuf.at[b3],
                                  isem.at[b3]),
        )

    def issue_idx1(k):
        for d in idx1_descs(k, lax.rem(k, 3)):
            d.start()

    def wait_idx1(k):
        for d in idx1_descs(k, lax.rem(k, 3)):
            d.wait()

    def p1scat_desc(k):
        b3 = lax.rem(k, 3)
        return pltpu.make_async_copy(tbuf.at[b3], tds.at[rbuf.at[b3]], psem)

    issue_idx1(0)
    issue_idx1(1)

    def p1(k, carry):
        wait_idx1(k)
        p1scat_desc(k).start(add=True)

        @pl.when(k >= 1)
        def _drain_prev():
            p1scat_desc(k - 1).wait()

        @pl.when(k + 2 < NCHUNK_B)
        def _next_idx():
            issue_idx1(k + 2)

        return carry

    lax.fori_loop(0, NCHUNK_B, p1, 0)
    p1scat_desc(NCHUNK_B - 1).wait()
    plsc.subcore_barrier()

    pltpu.sync_copy(tds, tdl)

    def fix(i, carry):
        sl = pl.ds(i * LANES, LANES)
        v = tdl[sl]
        tdl[sl] = jnp.where(v == 0.0, 1.0, v)
        return carry

    lax.fori_loop(0, NPAD // LANES, fix, 0)

    def idx3_descs(k, b3):
        base = s * EPT_B + k * K
        return (
            pltpu.make_async_copy(row_hbm.at[pl.ds(base, K)], rbuf.at[b3],
                                  isem.at[b3]),
            pltpu.make_async_copy(col_hbm.at[pl.ds(base, K)], cbuf.at[b3],
                                  isem.at[b3]),
            pltpu.make_async_copy(td_hbm.at[pl.ds(base, K)], tbuf.at[b3],
                                  isem.at[b3]),
        )

    def issue_idx3(k):
        for d in idx3_descs(k, lax.rem(k, 3)):
            d.start()

    def wait_idx3(k):
        for d in idx3_descs(k, lax.rem(k, 3)):
            d.wait()

    def gather_desc(k):
        b2 = lax.rem(k, 2)
        b3 = lax.rem(k, 3)
        return pltpu.make_async_copy(
            zwn_hbm.at[1].at[c].at[rbuf.at[b3]], rows.at[b2], gsem.at[b2])

    def scat_desc(k):
        b2 = lax.rem(k, 2)
        b3 = lax.rem(k, 3)
        return pltpu.make_async_copy(
            rows.at[b2], acc.at[cbuf.at[b3]], ssem.at[b2])

    issue_idx3(0)
    wait_idx3(0)
    gather_desc(0).start()
    issue_idx3(1)

    def chunk3(k, carry):
        b3 = lax.rem(k, 3)

        @pl.when(k >= 1)
        def _drain_scatter():
            scat_desc(k - 1).wait()

        @pl.when(k + 1 < NCHUNK_B)
        def _next_gather():
            wait_idx3(k + 1)
            gather_desc(k + 1).start()

        @pl.when(k + 2 < NCHUNK_B)
        def _next_idx():
            issue_idx3(k + 2)

        @plsc.parallel_loop(0, K // LANES, unroll=5)
        def relw(j):
            sl = pl.ds(j * LANES, LANES)
            den = plsc.load_gather(tdl, [rbuf[b3, sl]])
            tbuf[b3, sl] = tbuf[b3, sl] / den
        gather_desc(k).wait()
        b2 = lax.rem(k, 2)

        @plsc.parallel_loop(0, K, unroll=8)
        def scale(e):
            w = plsc.load_gather(
                tbuf, [jnp.full((LANES,), b3, jnp.int32),
                       jnp.full((LANES,), e, jnp.int32)])
            for j in range(JV):
                sl = pl.ds(j * LANES, LANES)
                rows[b2, e, sl] = rows[b2, e, sl] * w
        scat_desc(k).start(add=True)
        return carry

    lax.fori_loop(0, NCHUNK_B, chunk3, 0)
    scat_desc(NCHUNK_B - 1).wait()
    plsc.subcore_barrier()
    pltpu.sync_copy(acc.at[pl.ds(s * NPT, NPT)],
                    out_hbm.at[c].at[pl.ds(s * NPT, NPT)])


def kernel(z, edge_index, temporal_features, time_diffs, unique_edges,
           WS1, bS1, WT, bT, WS2, bS2, WN, bN, F1W, F1b, F2W, F2b):
    wcat = jnp.concatenate([WS1, WS2, WT, WN], axis=1)
    bcat = jnp.concatenate([bS1, bS2, bT, bN]).reshape(1, 4 * DH)
    selfs, gt = _pre_matmul(z, wcat, bcat)

    zeros = jnp.zeros((NPT, HALF), jnp.float32)
    zcol = jnp.zeros((NPT,), jnp.float32)

    out_t2 = _sc_temporal(unique_edges[0], unique_edges[1], gt,
                          temporal_features, zeros)

    td = time_diffs.astype(jnp.float32)
    out_s2 = _sc_structural(edge_index[0], edge_index[1], td, gt,
                            zeros, zcol)

    F1A = F1W[:DH]
    F1B = F1W[DH:]
    return _post(z, selfs, out_t2, out_s2,
                 F1A, F1B, F1b.reshape(1, DIN), F2W, F2b.reshape(1, DIN))


# speedup vs baseline: 1.9122x; 1.0684x over previous
import functools

import jax
import jax.numpy as jnp
from jax import lax
from jax.experimental import pallas as pl
from jax.experimental.pallas import tpu as pltpu
from jax.experimental.pallas import tpu_sc as plsc

N = 10000
E = 320000
EU = 160000
DIN = 128
DH = 256

NC = 2
NS = 16
LANES = 16

ROWS_BLK = 1000
HALF = DH // NC
NPAD = 10240
NPT = NPAD // NS
K = 80
JV = HALF // LANES

_MESH = plsc.VectorSubcoreMesh(core_axis_name="c", subcore_axis_name="s",
                               num_cores=NC, num_subcores=NS)
_SC_PARAMS = pltpu.CompilerParams(needs_layout_passes=False)


def _pre_matmul_body(z_ref, w_ref, b_ref, selfs_ref, gt_ref):
    x = (jnp.dot(z_ref[...], w_ref[...], preferred_element_type=jnp.float32)
         + b_ref[...])
    selfs_ref[...] = x[:, :2 * DH]
    gt_ref[0, 0] = x[:, 2 * DH:2 * DH + HALF]
    gt_ref[0, 1] = x[:, 2 * DH + HALF:3 * DH]
    gt_ref[1, 0] = x[:, 3 * DH:3 * DH + HALF]
    gt_ref[1, 1] = x[:, 3 * DH + HALF:4 * DH]


def _pre_matmul(z, wcat, bcat):
    grid = (N // ROWS_BLK,)
    return pl.pallas_call(
        _pre_matmul_body,
        grid=grid,
        in_specs=[
            pl.BlockSpec((ROWS_BLK, DIN), lambda i: (i, 0)),
            pl.BlockSpec((DIN, 4 * DH), lambda i: (0, 0)),
            pl.BlockSpec((1, 4 * DH), lambda i: (0, 0)),
        ],
        out_specs=[
            pl.BlockSpec((ROWS_BLK, 2 * DH), lambda i: (i, 0)),
            pl.BlockSpec((2, 2, ROWS_BLK, HALF), lambda i: (0, 0, i, 0)),
        ],
        out_shape=[
            jax.ShapeDtypeStruct((N, 2 * DH), jnp.float32),
            jax.ShapeDtypeStruct((2, 2, N, HALF), jnp.float32),
        ],
    )(z, wcat, bcat)


def _normalize_rows(x):
    n = jnp.sqrt(jnp.sum(x * x, axis=1, keepdims=True))
    return x / jnp.maximum(n, 1e-12)


def _post_body(z_ref, selfs_ref, ot_ref, os_ref,
               f1a_ref, f1b_ref, f1bias_ref, f2w_ref, f2bias_ref, out_ref):
    sf = selfs_ref[:, :DH]
    sf2 = selfs_ref[:, DH:]
    ot = jnp.concatenate([ot_ref[0], ot_ref[1]], axis=-1)
    os_ = jnp.concatenate([os_ref[0], os_ref[1]], axis=-1)
    r = _normalize_rows(jax.nn.relu(sf + ot))
    g = _normalize_rows(jax.nn.relu(sf2 + os_))
    h1 = jax.nn.relu(
        jnp.dot(r, f1a_ref[...], preferred_element_type=jnp.float32)
        + jnp.dot(g, f1b_ref[...], preferred_element_type=jnp.float32)
        + f1bias_ref[...]
    )
    h = jnp.dot(h1, f2w_ref[...], preferred_element_type=jnp.float32) + f2bias_ref[...]
    out_ref[...] = _normalize_rows(z_ref[...] + jax.nn.relu(h))


def _post(z, selfs, out_t2, out_s2, F1A, F1B, F1bias, F2W, F2bias):
    grid = (N // ROWS_BLK,)
    full_spec = lambda r, c: pl.BlockSpec((r, c), lambda i: (0, 0))
    return pl.pallas_call(
        _post_body,
        grid=grid,
        in_specs=[
            pl.BlockSpec((ROWS_BLK, DIN), lambda i: (i, 0)),
            pl.BlockSpec((ROWS_BLK, 2 * DH), lambda i: (i, 0)),
            pl.BlockSpec((2, ROWS_BLK, HALF), lambda i: (0, i, 0)),
            pl.BlockSpec((2, ROWS_BLK, HALF), lambda i: (0, i, 0)),
            full_spec(DH, DIN), full_spec(DH, DIN), full_spec(1, DIN),
            full_spec(DIN, DIN), full_spec(1, DIN),
        ],
        out_specs=pl.BlockSpec((ROWS_BLK, DIN), lambda i: (i, 0)),
        out_shape=jax.ShapeDtypeStruct((N, DIN), jnp.float32),
    )(z, selfs, out_t2, out_s2, F1A, F1B, F1bias, F2W, F2bias)


EPT_A = EU // NS
NCHUNK_A = EPT_A // K


@functools.partial(
    pl.kernel,
    out_type=jax.ShapeDtypeStruct((NC, NPAD, HALF), jnp.float32),
    mesh=_MESH,
    compiler_params=_SC_PARAMS,
    scratch_types=[
        pltpu.VMEM_SHARED((NPAD, HALF), jnp.float32),
        pltpu.VMEM((3, K), jnp.int32),
        pltpu.VMEM((3, K), jnp.int32),
        pltpu.VMEM((2, K, HALF), jnp.float32),
        pltpu.VMEM((2, K, HALF), jnp.float32),
        pltpu.SemaphoreType.DMA((3,)),
        pltpu.SemaphoreType.DMA((2,)),
        pltpu.SemaphoreType.DMA((2,)),
        pltpu.SemaphoreType.DMA((2,)),
        pltpu.SemaphoreType.DMA((2,)),
    ],
)
def _sc_temporal(src_hbm, dst_hbm, zwt_hbm, tf_hbm, zeros_hbm, out_hbm,
                 acc, sbuf, dbuf, rows, tfb, isem, gsem, tsem, ssem, tssem):
    c = lax.axis_index("c")
    s = lax.axis_index("s")

    pltpu.sync_copy(zeros_hbm, acc.at[pl.ds(s * NPT, NPT)])
    plsc.subcore_barrier()

    def idx_descs(k, b3):
        base = s * EPT_A + k * K
        return (
            pltpu.make_async_copy(src_hbm.at[pl.ds(base, K)], sbuf.at[b3],
                                  isem.at[b3]),
            pltpu.make_async_copy(dst_hbm.at[pl.ds(base, K)], dbuf.at[b3],
                                  isem.at[b3]),
        )

    def issue_idx(k):
        for d in idx_descs(k, lax.rem(k, 3)):
            d.start()

    def wait_idx(k):
        for d in idx_descs(k, lax.rem(k, 3)):
            d.wait()

    def gather_desc(k):
        b2 = lax.rem(k, 2)
        b3 = lax.rem(k, 3)
        return pltpu.make_async_copy(
            zwt_hbm.at[0].at[c].at[sbuf.at[b3]], rows.at[b2], gsem.at[b2])

    def tf_desc(k):
        b2 = lax.rem(k, 2)
        base = s * EPT_A + k * K
        return pltpu.make_async_copy(
            tf_hbm.at[pl.ds(base, K), pl.ds(c * HALF, HALF)],
            tfb.at[b2], tsem.at[b2])

    def scat_desc(k):
        b2 = lax.rem(k, 2)
        b3 = lax.rem(k, 3)
        return pltpu.make_async_copy(
            rows.at[b2], acc.at[dbuf.at[b3]], ssem.at[b2])

    def scat_tf_desc(k):
        b2 = lax.rem(k, 2)
        b3 = lax.rem(k, 3)
        return pltpu.make_async_copy(
            tfb.at[b2], acc.at[dbuf.at[b3]], tssem.at[b2])

    issue_idx(0)
    wait_idx(0)
    gather_desc(0).start()
    tf_desc(0).start()
    issue_idx(1)

    def chunk(k, carry):
        b2 = lax.rem(k, 2)

        @pl.when(k >= 1)
        def _drain_scatter():
            scat_desc(k - 1).wait()
            scat_tf_desc(k - 1).wait()

        @pl.when(k + 1 < NCHUNK_A)
        def _next_streams():
            wait_idx(k + 1)
            gather_desc(k + 1).start()
            tf_desc(k + 1).start()

        @pl.when(k + 2 < NCHUNK_A)
        def _next_idx():
            issue_idx(k + 2)

        gather_desc(k).wait()
        scat_desc(k).start(add=True)
        tf_desc(k).wait()
        scat_tf_desc(k).start(add=True)
        return carry

    lax.fori_loop(0, NCHUNK_A, chunk, 0)
    scat_desc(NCHUNK_A - 1).wait()
    scat_tf_desc(NCHUNK_A - 1).wait()
    plsc.subcore_barrier()
    pltpu.sync_copy(acc.at[pl.ds(s * NPT, NPT)],
                    out_hbm.at[c].at[pl.ds(s * NPT, NPT)])


KB = 128
NCHUNK_TOT = E // KB
BASE_B = NCHUNK_TOT // NS
EXTRA_B = NCHUNK_TOT - BASE_B * NS


@functools.partial(
    pl.kernel,
    out_type=jax.ShapeDtypeStruct((NC, NPAD, HALF), jnp.float32),
    mesh=_MESH,
    compiler_params=_SC_PARAMS,
    scratch_types=[
        pltpu.VMEM_SHARED((NPAD, HALF), jnp.float32),
        pltpu.VMEM_SHARED((NPAD,), jnp.float32),
        pltpu.VMEM((NPAD,), jnp.float32),
        pltpu.VMEM((3, KB), jnp.int32),
        pltpu.VMEM((3, KB), jnp.int32),
        pltpu.VMEM((3, KB), jnp.float32),
        pltpu.VMEM((2, KB, HALF), jnp.float32),
        pltpu.SemaphoreType.DMA((3,)),
        pltpu.SemaphoreType.DMA((2,)),
        pltpu.SemaphoreType.DMA,
        pltpu.SemaphoreType.DMA((2,)),
    ],
)
def _sc_structural(row_hbm, col_hbm, td_hbm, zwn_hbm, zeros_hbm, zcol_hbm,
                   out_hbm, acc, tds, tdl, rbuf, cbuf, tbuf, rows,
                   isem, gsem, psem, ssem):
    c = lax.axis_index("c")
    s = lax.axis_index("s")
    nck = BASE_B + jnp.where(s < EXTRA_B, 1, 0)
    tbase = s * (BASE_B * KB) + jnp.minimum(s, EXTRA_B) * KB

    pltpu.sync_copy(zeros_hbm, acc.at[pl.ds(s * NPT, NPT)])
    pltpu.sync_copy(zcol_hbm, tds.at[pl.ds(s * NPT, NPT)])
    plsc.subcore_barrier()

    def idx1_descs(k, b3):
        base = tbase + k * KB
        return (
            pltpu.make_async_copy(row_hbm.at[pl.ds(base, KB)], rbuf.at[b3],
                                  isem.at[b3]),
            pltpu.make_async_copy(td_hbm.at[pl.ds(base, KB)], tbuf.at[b3],
                                  isem.at[b3]),
        )

    def issue_idx1(k):
        for d in idx1_descs(k, lax.rem(k, 3)):
            d.start()

    def wait_idx1(k):
        for d in idx1_descs(k, lax.rem(k, 3)):
            d.wait()

    def p1scat_desc(k):
        b3 = lax.rem(k, 3)
        return pltpu.make_async_copy(tbuf.at[b3], tds.at[rbuf.at[b3]], psem)

    issue_idx1(0)
    issue_idx1(1)

    def p1(k, carry):
        wait_idx1(k)
        p1scat_desc(k).start(add=True)

        @pl.when(k >= 1)
        def _drain_prev():
            p1scat_desc(k - 1).wait()

        @pl.when(k + 2 < nck)
        def _next_idx():
            issue_idx1(k + 2)

        return carry

    lax.fori_loop(0, nck, p1, 0)
    p1scat_desc(nck - 1).wait()
    plsc.subcore_barrier()

    pltpu.sync_copy(tds, tdl)

    def fix(i, carry):
        sl = pl.ds(i * LANES, LANES)
        v = tdl[sl]
        tdl[sl] = jnp.where(v == 0.0, 1.0, v)
        return carry

    lax.fori_loop(0, NPAD // LANES, fix, 0)

    def idx3_descs(k, b3):
        base = tbase + k * KB
        return (
            pltpu.make_async_copy(row_hbm.at[pl.ds(base, KB)], rbuf.at[b3],
                                  isem.at[b3]),
            pltpu.make_async_copy(col_hbm.at[pl.ds(base, KB)], cbuf.at[b3],
                                  isem.at[b3]),
            pltpu.make_async_copy(td_hbm.at[pl.ds(base, KB)], tbuf.at[b3],
                                  isem.at[b3]),
        )

    def issue_idx3(k):
        for d in idx3_descs(k, lax.rem(k, 3)):
            d.start()

    def wait_idx3(k):
        for d in idx3_descs(k, lax.rem(k, 3)):
            d.wait()

    def gather_desc(k):
        b2 = lax.rem(k, 2)
        b3 = lax.rem(k, 3)
        return pltpu.make_async_copy(
            zwn_hbm.at[1].at[c].at[rbuf.at[b3]], rows.at[b2], gsem.at[b2])

    def scat_desc(k):
        b2 = lax.rem(k, 2)
        b3 = lax.rem(k, 3)
        return pltpu.make_async_copy(
            rows.at[b2], acc.at[cbuf.at[b3]], ssem.at[b2])

    issue_idx3(0)
    wait_idx3(0)
    gather_desc(0).start()
    issue_idx3(1)

    def chunk3(k, carry):
        b3 = lax.rem(k, 3)

        @pl.when(k >= 1)
        def _drain_scatter():
            scat_desc(k - 1).wait()

        @pl.when(k + 1 < nck)
        def _next_gather():
            wait_idx3(k + 1)
            gather_desc(k + 1).start()

        @pl.when(k + 2 < nck)
        def _next_idx():
            issue_idx3(k + 2)

        @plsc.parallel_loop(0, KB // LANES, unroll=8)
        def relw(j):
            sl = pl.ds(j * LANES, LANES)
            den = plsc.load_gather(tdl, [rbuf[b3, sl]])
            tbuf[b3, sl] = tbuf[b3, sl] / den
        gather_desc(k).wait()
        b2 = lax.rem(k, 2)

        @plsc.parallel_loop(0, KB, unroll=8)
        def scale(e):
            w = plsc.load_gather(
                tbuf, [jnp.full((LANES,), b3, jnp.int32),
                       jnp.full((LANES,), e, jnp.int32)])
            for j in range(JV):
                sl = pl.ds(j * LANES, LANES)
                rows[b2, e, sl] = rows[b2, e, sl] * w
        scat_desc(k).start(add=True)
        return carry

    lax.fori_loop(0, nck, chunk3, 0)
    scat_desc(nck - 1).wait()
    plsc.subcore_barrier()
    pltpu.sync_copy(acc.at[pl.ds(s * NPT, NPT)],
                    out_hbm.at[c].at[pl.ds(s * NPT, NPT)])


def kernel(z, edge_index, temporal_features, time_diffs, unique_edges,
           WS1, bS1, WT, bT, WS2, bS2, WN, bN, F1W, F1b, F2W, F2b):
    wcat = jnp.concatenate([WS1, WS2, WT, WN], axis=1)
    bcat = jnp.concatenate([bS1, bS2, bT, bN]).reshape(1, 4 * DH)
    selfs, gt = _pre_matmul(z, wcat, bcat)

    zeros = jnp.zeros((NPT, HALF), jnp.float32)
    zcol = jnp.zeros((NPT,), jnp.float32)

    out_t2 = _sc_temporal(unique_edges[0], unique_edges[1], gt,
                          temporal_features, zeros)

    td = time_diffs.astype(jnp.float32)
    out_s2 = _sc_structural(edge_index[0], edge_index[1], td, gt,
                            zeros, zcol)

    F1A = F1W[:DH]
    F1B = F1W[DH:]
    return _post(z, selfs, out_t2, out_s2,
                 F1A, F1B, F1b.reshape(1, DIN), F2W, F2b.reshape(1, DIN))
